# Initial kernel scaffold; baseline (speedup 1.0000x reference)
import functools

import jax
import jax.numpy as jnp
from jax import lax
from jax.experimental import pallas as pl
from jax.experimental.pallas import tpu as pltpu
from jax.experimental.pallas import tpu_sc as plsc

_F32 = jnp.float32
_NC = 2
_NS = 16
_NW = _NC * _NS
_L = 16


def _mesh():
    return plsc.VectorSubcoreMesh(core_axis_name="c", subcore_axis_name="s")


@functools.lru_cache(maxsize=None)
def _make_sc_gather(NP, E):
    EC = E // _NW

    @functools.partial(
        pl.kernel,
        mesh=_mesh(),
        out_type=tuple(jax.ShapeDtypeStruct((E,), _F32) for _ in range(4)),
        scratch_types=(
            [pltpu.VMEM((NP,), _F32)] * 3
            + [pltpu.VMEM((EC,), jnp.int32)] * 2
            + [pltpu.VMEM((EC,), _F32)] * 4
        ),
    )
    def k(x0, x1, x2, row, col, dx, dy, dz, r2,
          x0v, x1v, x2v, rowv, colv, dxv, dyv, dzv, r2v):
        wid = lax.axis_index("c") * _NS + lax.axis_index("s")
        base = wid * EC
        pltpu.sync_copy(x0, x0v)
        pltpu.sync_copy(x1, x1v)
        pltpu.sync_copy(x2, x2v)
        pltpu.sync_copy(row.at[pl.ds(base, EC)], rowv)
        pltpu.sync_copy(col.at[pl.ds(base, EC)], colv)

        def body(i, carry):
            s = pl.ds(i * _L, _L)
            rv = rowv[s]
            cv = colv[s]
            ax = plsc.load_gather(x0v, [rv]) - plsc.load_gather(x0v, [cv])
            ay = plsc.load_gather(x1v, [rv]) - plsc.load_gather(x1v, [cv])
            az = plsc.load_gather(x2v, [rv]) - plsc.load_gather(x2v, [cv])
            dxv[s] = ax
            dyv[s] = ay
            dzv[s] = az
            r2v[s] = ax * ax + ay * ay + az * az
            return carry

        lax.fori_loop(0, EC // _L, body, 0, unroll=4)
        pltpu.sync_copy(dxv, dx.at[pl.ds(base, EC)])
        pltpu.sync_copy(dyv, dy.at[pl.ds(base, EC)])
        pltpu.sync_copy(dzv, dz.at[pl.ds(base, EC)])
        pltpu.sync_copy(r2v, r2.at[pl.ds(base, EC)])

    return k


@functools.lru_cache(maxsize=None)
def _make_sc_scatter(NP, CH, CB):
    @functools.partial(
        pl.kernel,
        mesh=_mesh(),
        out_type=jax.ShapeDtypeStruct((_NC, 4, NP), _F32),
        scratch_types=(
            [pltpu.VMEM((CH, CB), jnp.int32)]
            + [pltpu.VMEM((CH, CB), _F32)] * 4
            + [pltpu.VMEM_SHARED((NP,), _F32)] * 4
        ),
    )
    def k(row3, mx3, my3, mz3, ones2, zerosN, out,
          rowv, mxv, myv, mzv, onev, accx, accy, accz, accn):
        cid = lax.axis_index("c")
        sid = lax.axis_index("s")
        wid = cid * _NS + sid

        @pl.when(sid == 0)
        def _zero():
            pltpu.sync_copy(zerosN, accx)
            pltpu.sync_copy(zerosN, accy)
            pltpu.sync_copy(zerosN, accz)
            pltpu.sync_copy(zerosN, accn)

        pltpu.sync_copy(row3.at[wid], rowv)
        pltpu.sync_copy(mx3.at[wid], mxv)
        pltpu.sync_copy(my3.at[wid], myv)
        pltpu.sync_copy(mz3.at[wid], mzv)
        pltpu.sync_copy(ones2, onev)
        plsc.subcore_barrier()

        def body(j, carry):
            idx = rowv.at[j]
            pltpu.sync_copy(mxv.at[j], accx.at[idx], add=True)
            pltpu.sync_copy(myv.at[j], accy.at[idx], add=True)
            pltpu.sync_copy(mzv.at[j], accz.at[idx], add=True)
            pltpu.sync_copy(onev.at[j], accn.at[idx], add=True)
            return carry

        lax.fori_loop(0, CH, body, 0)
        plsc.subcore_barrier()

        @pl.when(sid == 0)
        def _o0():
            pltpu.sync_copy(accx, out.at[cid, 0])

        @pl.when(sid == 1)
        def _o1():
            pltpu.sync_copy(accy, out.at[cid, 1])

        @pl.when(sid == 2)
        def _o2():
            pltpu.sync_copy(accz, out.at[cid, 2])

        @pl.when(sid == 3)
        def _o3():
            pltpu.sync_copy(accn, out.at[cid, 3])

    return k


@functools.lru_cache(maxsize=None)
def _make_sc_update(NP):
    CN = NP // _NW

    @functools.partial(
        pl.kernel,
        mesh=_mesh(),
        out_type=jax.ShapeDtypeStruct((3, NP), _F32),
        scratch_types=(
            [pltpu.VMEM((3, CN), _F32)] * 2
            + [pltpu.VMEM((CN,), _F32)]
            + [pltpu.VMEM((4, CN), _F32)] * 2
            + [pltpu.VMEM((3, CN), _F32)]
        ),
    )
    def k(xp, vp, vsl, parts, out, xv, vv, vsv, p0v, p1v, ov):
        wid = lax.axis_index("c") * _NS + lax.axis_index("s")
        base = wid * CN
        pltpu.sync_copy(xp.at[:, pl.ds(base, CN)], xv)
        pltpu.sync_copy(vp.at[:, pl.ds(base, CN)], vv)
        pltpu.sync_copy(vsl.at[pl.ds(base, CN)], vsv)
        pltpu.sync_copy(parts.at[0, :, pl.ds(base, CN)], p0v)
        pltpu.sync_copy(parts.at[1, :, pl.ds(base, CN)], p1v)

        def body(i, carry):
            s = pl.ds(i * _L, _L)
            cnt = p0v[3, s] + p1v[3, s]
            inv = 1.0 / jnp.maximum(cnt, 1.0)
            vs = vsv[s]
            for c in range(3):
                ov[c, s] = xv[c, s] + (p0v[c, s] + p1v[c, s]) * inv + vv[c, s] * vs
            return carry

        lax.fori_loop(0, CN // _L, body, 0, unroll=2)
        pltpu.sync_copy(ov, out.at[:, pl.ds(base, CN)])

    return k


def _tc_vscale(vp, wv0t, bv0c, wv1, bv1c):
    NL, HID = wv0t.shape[0], wv0t.shape[1]
    NP = vp.shape[1]

    def body(vp_ref, w0_ref, b0_ref, w1_ref, b1_ref, o_ref):
        v0 = vp_ref[0:1, :]
        v1 = vp_ref[1:2, :]
        v2 = vp_ref[2:3, :]
        vn = jnp.sqrt(v0 * v0 + v1 * v1 + v2 * v2)
        hid = w0_ref[0] * vn + b0_ref[0]
        hid = hid * jax.nn.sigmoid(hid)
        vs = jnp.sum(hid * w1_ref[0], axis=0, keepdims=True) + b1_ref[0]
        o_ref[...] = vs

    return pl.pallas_call(
        body,
        grid=(NL,),
        in_specs=[
            pl.BlockSpec((3, NP), lambda l: (0, 0)),
            pl.BlockSpec((1, HID, 1), lambda l: (l, 0, 0)),
            pl.BlockSpec((1, HID, 1), lambda l: (l, 0, 0)),
            pl.BlockSpec((1, HID, 1), lambda l: (l, 0, 0)),
            pl.BlockSpec((1, 1, 1), lambda l: (l, 0, 0)),
        ],
        out_specs=pl.BlockSpec((1, NP), lambda l: (l, 0)),
        out_shape=jax.ShapeDtypeStruct((NL, NP), _F32),
    )(vp, wv0t, bv0c, wv1, bv1c)


def _tc_edge_mlp(eaT, r2r, dxr, dyr, dzr, w0r, w0e, b0c, w1c):
    D_EDGE, E = eaT.shape
    HID = w0e.shape[0]
    BE = 2560
    G = E // BE

    def body(ea_ref, r2_ref, dx_ref, dy_ref, dz_ref,
             w0r_ref, w0e_ref, b0_ref, w1_ref, mx_ref, my_ref, mz_ref):
        rad = jnp.sqrt(r2_ref[...])
        hid = lax.dot_general(
            w0e_ref[...], ea_ref[...],
            (((1,), (0,)), ((), ())),
            preferred_element_type=_F32)
        hid = hid + w0r_ref[...] * rad + b0_ref[...]
        hid = hid * jax.nn.sigmoid(hid)
        t = jnp.tanh(jnp.sum(hid * w1_ref[...], axis=0, keepdims=True))
        mx_ref[...] = dx_ref[...] * t
        my_ref[...] = dy_ref[...] * t
        mz_ref[...] = dz_ref[...] * t

    return pl.pallas_call(
        body,
        grid=(G,),
        in_specs=[
            pl.BlockSpec((D_EDGE, BE), lambda j: (0, j)),
            pl.BlockSpec((1, BE), lambda j: (0, j)),
            pl.BlockSpec((1, BE), lambda j: (0, j)),
            pl.BlockSpec((1, BE), lambda j: (0, j)),
            pl.BlockSpec((1, BE), lambda j: (0, j)),
            pl.BlockSpec((HID, 1), lambda j: (0, 0)),
            pl.BlockSpec((HID, D_EDGE), lambda j: (0, 0)),
            pl.BlockSpec((HID, 1), lambda j: (0, 0)),
            pl.BlockSpec((HID, 1), lambda j: (0, 0)),
        ],
        out_specs=[pl.BlockSpec((1, BE), lambda j: (0, j))] * 3,
        out_shape=tuple(jax.ShapeDtypeStruct((1, E), _F32) for _ in range(3)),
    )(eaT, r2r, dxr, dyr, dzr, w0r, w0e, b0c, w1c)


def kernel(x, h, v, edge_attr, edge_index,
           W_phi0, b_phi0, W_phi1, W_vel0, b_vel0, W_vel1, b_vel1):
    N = x.shape[0]
    E = edge_attr.shape[0]
    NL = W_phi0.shape[0]
    NP = ((N + 8 * _NW - 1) // (8 * _NW)) * (8 * _NW)
    row = edge_index[0]
    col = edge_index[1]

    xp = jnp.zeros((3, NP), _F32).at[:, :N].set(x.T)
    vp = jnp.zeros((3, NP), _F32).at[:, :N].set(v.T)
    eaT = edge_attr.T
    CB = 125
    CH = (E // _NW) // CB
    row3 = row.reshape(_NW, CH, CB)
    ones2 = jnp.ones((CH, CB), _F32)
    zerosN = jnp.zeros((NP,), _F32)

    w0r = W_phi0[:, 0:1, :].transpose(0, 2, 1)
    w0e = W_phi0[:, 1:, :].transpose(0, 2, 1)
    b0c = b_phi0[:, :, None]
    w1c = W_phi1
    wv0t = W_vel0.transpose(0, 2, 1)
    bv0c = b_vel0[:, :, None]
    bv1c = b_vel1[:, :, None]

    vs_all = _tc_vscale(vp, wv0t, bv0c, W_vel1, bv1c)

    gather_k = _make_sc_gather(NP, E)
    scat_k = _make_sc_scatter(NP, CH, CB)
    upd_k = _make_sc_update(NP)

    for l in range(NL):
        dx, dy, dz, r2 = gather_k(xp[0], xp[1], xp[2], row, col)
        mx, my, mz = _tc_edge_mlp(
            eaT, r2.reshape(1, E), dx.reshape(1, E), dy.reshape(1, E),
            dz.reshape(1, E), w0r[l], w0e[l], b0c[l], w1c[l])
        parts = scat_k(
            row3, mx.reshape(_NW, CH, CB), my.reshape(_NW, CH, CB),
            mz.reshape(_NW, CH, CB), ones2, zerosN)
        xp = upd_k(xp, vp, vs_all[l], parts)

    xout = xp[:, :N].T
    return xout, h


# speedup vs baseline: 9.8812x; 9.8812x over previous
import functools

import jax
import jax.numpy as jnp
from jax import lax
from jax.experimental import pallas as pl
from jax.experimental.pallas import tpu as pltpu
from jax.experimental.pallas import tpu_sc as plsc

_F32 = jnp.float32
_NC = 2
_NS = 16
_NW = _NC * _NS
_L = 16


def _mesh():
    return plsc.VectorSubcoreMesh(
        core_axis_name="c", subcore_axis_name="s",
        num_cores=_NC, num_subcores=_NS)


_SC_PARAMS = pltpu.CompilerParams(needs_layout_passes=False)


@functools.lru_cache(maxsize=None)
def _make_sc_gather(NP, E):
    EC = E // _NW

    @functools.partial(
        pl.kernel,
        mesh=_mesh(),
        compiler_params=_SC_PARAMS,
        out_type=tuple(jax.ShapeDtypeStruct((E,), _F32) for _ in range(4)),
        scratch_types=(
            [pltpu.VMEM((NP,), _F32)] * 3
            + [pltpu.VMEM((EC,), jnp.int32)] * 2
            + [pltpu.VMEM((EC,), _F32)] * 4
        ),
    )
    def k(x0, x1, x2, row, col, dx, dy, dz, r2,
          x0v, x1v, x2v, rowv, colv, dxv, dyv, dzv, r2v):
        wid = lax.axis_index("c") * _NS + lax.axis_index("s")
        base = wid * EC
        pltpu.sync_copy(x0, x0v)
        pltpu.sync_copy(x1, x1v)
        pltpu.sync_copy(x2, x2v)
        pltpu.sync_copy(row.at[pl.ds(base, EC)], rowv)
        pltpu.sync_copy(col.at[pl.ds(base, EC)], colv)

        def body(i, carry):
            s = pl.ds(i * _L, _L)
            rv = rowv[s]
            cv = colv[s]
            ax = plsc.load_gather(x0v, [rv]) - plsc.load_gather(x0v, [cv])
            ay = plsc.load_gather(x1v, [rv]) - plsc.load_gather(x1v, [cv])
            az = plsc.load_gather(x2v, [rv]) - plsc.load_gather(x2v, [cv])
            dxv[s] = ax
            dyv[s] = ay
            dzv[s] = az
            r2v[s] = ax * ax + ay * ay + az * az
            return carry

        lax.fori_loop(0, EC // _L, body, 0, unroll=4)
        pltpu.sync_copy(dxv, dx.at[pl.ds(base, EC)])
        pltpu.sync_copy(dyv, dy.at[pl.ds(base, EC)])
        pltpu.sync_copy(dzv, dz.at[pl.ds(base, EC)])
        pltpu.sync_copy(r2v, r2.at[pl.ds(base, EC)])

    return k


@functools.lru_cache(maxsize=None)
def _make_sc_scatter(NP, CH, CB):
    @functools.partial(
        pl.kernel,
        mesh=_mesh(),
        compiler_params=_SC_PARAMS,
        out_type=jax.ShapeDtypeStruct((_NC, 4, NP), _F32),
        scratch_types=(
            [pltpu.VMEM((CH, CB), jnp.int32)]
            + [pltpu.VMEM((CH, CB), _F32)] * 4
            + [pltpu.VMEM_SHARED((NP,), _F32)] * 4
        ),
    )
    def k(row3, mx3, my3, mz3, ones2, zerosN, out,
          rowv, mxv, myv, mzv, onev, accx, accy, accz, accn):
        cid = lax.axis_index("c")
        sid = lax.axis_index("s")
        wid = cid * _NS + sid

        @pl.when(sid == 0)
        def _zero():
            pltpu.sync_copy(zerosN, accx)
            pltpu.sync_copy(zerosN, accy)
            pltpu.sync_copy(zerosN, accz)
            pltpu.sync_copy(zerosN, accn)

        pltpu.sync_copy(row3.at[wid], rowv)
        pltpu.sync_copy(mx3.at[wid], mxv)
        pltpu.sync_copy(my3.at[wid], myv)
        pltpu.sync_copy(mz3.at[wid], mzv)
        pltpu.sync_copy(ones2, onev)
        plsc.subcore_barrier()

        def body(j, carry):
            idx = rowv.at[j]
            pltpu.sync_copy(mxv.at[j], accx.at[idx], add=True)
            pltpu.sync_copy(myv.at[j], accy.at[idx], add=True)
            pltpu.sync_copy(mzv.at[j], accz.at[idx], add=True)
            pltpu.sync_copy(onev.at[j], accn.at[idx], add=True)
            return carry

        lax.fori_loop(0, CH, body, 0)
        plsc.subcore_barrier()

        @pl.when(sid == 0)
        def _o0():
            pltpu.sync_copy(accx, out.at[cid, 0])

        @pl.when(sid == 1)
        def _o1():
            pltpu.sync_copy(accy, out.at[cid, 1])

        @pl.when(sid == 2)
        def _o2():
            pltpu.sync_copy(accz, out.at[cid, 2])

        @pl.when(sid == 3)
        def _o3():
            pltpu.sync_copy(accn, out.at[cid, 3])

    return k


@functools.lru_cache(maxsize=None)
def _make_sc_update(NP):
    CN = NP // _NW

    @functools.partial(
        pl.kernel,
        mesh=_mesh(),
        compiler_params=_SC_PARAMS,
        out_type=jax.ShapeDtypeStruct((3, NP), _F32),
        scratch_types=(
            [pltpu.VMEM((3, CN), _F32)] * 2
            + [pltpu.VMEM((CN,), _F32)]
            + [pltpu.VMEM((4, CN), _F32)] * 2
            + [pltpu.VMEM((3, CN), _F32)]
        ),
    )
    def k(xp, vp, vsl, parts, out, xv, vv, vsv, p0v, p1v, ov):
        wid = lax.axis_index("c") * _NS + lax.axis_index("s")
        base = wid * CN
        pltpu.sync_copy(xp.at[:, pl.ds(base, CN)], xv)
        pltpu.sync_copy(vp.at[:, pl.ds(base, CN)], vv)
        pltpu.sync_copy(vsl.at[pl.ds(base, CN)], vsv)
        pltpu.sync_copy(parts.at[0, :, pl.ds(base, CN)], p0v)
        pltpu.sync_copy(parts.at[1, :, pl.ds(base, CN)], p1v)

        def body(i, carry):
            s = pl.ds(i * _L, _L)
            cnt = p0v[3, s] + p1v[3, s]
            inv = 1.0 / jnp.maximum(cnt, 1.0)
            vs = vsv[s]
            for c in range(3):
                ov[c, s] = xv[c, s] + (p0v[c, s] + p1v[c, s]) * inv + vv[c, s] * vs
            return carry

        lax.fori_loop(0, CN // _L, body, 0, unroll=2)
        pltpu.sync_copy(ov, out.at[:, pl.ds(base, CN)])

    return k


def _tc_vscale(vp, wv0t, bv0c, wv1, bv1c):
    NL, HID = wv0t.shape[0], wv0t.shape[1]
    NP = vp.shape[1]

    def body(vp_ref, w0_ref, b0_ref, w1_ref, b1_ref, o_ref):
        v0 = vp_ref[0:1, :]
        v1 = vp_ref[1:2, :]
        v2 = vp_ref[2:3, :]
        vn = jnp.sqrt(v0 * v0 + v1 * v1 + v2 * v2)
        hid = w0_ref[0] * vn + b0_ref[0]
        hid = hid * jax.nn.sigmoid(hid)
        vs = jnp.sum(hid * w1_ref[0], axis=0, keepdims=True) + b1_ref[0]
        o_ref[0] = vs

    return pl.pallas_call(
        body,
        grid=(NL,),
        in_specs=[
            pl.BlockSpec((3, NP), lambda l: (0, 0)),
            pl.BlockSpec((1, HID, 1), lambda l: (l, 0, 0)),
            pl.BlockSpec((1, HID, 1), lambda l: (l, 0, 0)),
            pl.BlockSpec((1, HID, 1), lambda l: (l, 0, 0)),
            pl.BlockSpec((1, 1, 1), lambda l: (l, 0, 0)),
        ],
        out_specs=pl.BlockSpec((1, 1, NP), lambda l: (l, 0, 0)),
        out_shape=jax.ShapeDtypeStruct((NL, 1, NP), _F32),
    )(vp, wv0t, bv0c, wv1, bv1c).reshape(NL, NP)


def _tc_edge_mlp(eaT, r2r, dxr, dyr, dzr, w0r, w0e, b0c, w1c):
    D_EDGE, E = eaT.shape
    HID = w0e.shape[0]
    BE = 2048
    G = E // BE

    def body(ea_ref, r2_ref, dx_ref, dy_ref, dz_ref,
             w0r_ref, w0e_ref, b0_ref, w1_ref, mx_ref, my_ref, mz_ref):
        rad = jnp.sqrt(r2_ref[...])
        hid = lax.dot_general(
            w0e_ref[...], ea_ref[...],
            (((1,), (0,)), ((), ())),
            preferred_element_type=_F32)
        hid = hid + w0r_ref[...] * rad + b0_ref[...]
        hid = hid * jax.nn.sigmoid(hid)
        t = jnp.tanh(jnp.sum(hid * w1_ref[...], axis=0, keepdims=True))
        mx_ref[...] = dx_ref[...] * t
        my_ref[...] = dy_ref[...] * t
        mz_ref[...] = dz_ref[...] * t

    return pl.pallas_call(
        body,
        grid=(G,),
        in_specs=[
            pl.BlockSpec((D_EDGE, BE), lambda j: (0, j)),
            pl.BlockSpec((1, BE), lambda j: (0, j)),
            pl.BlockSpec((1, BE), lambda j: (0, j)),
            pl.BlockSpec((1, BE), lambda j: (0, j)),
            pl.BlockSpec((1, BE), lambda j: (0, j)),
            pl.BlockSpec((HID, 1), lambda j: (0, 0)),
            pl.BlockSpec((HID, D_EDGE), lambda j: (0, 0)),
            pl.BlockSpec((HID, 1), lambda j: (0, 0)),
            pl.BlockSpec((HID, 1), lambda j: (0, 0)),
        ],
        out_specs=[pl.BlockSpec((1, BE), lambda j: (0, j))] * 3,
        out_shape=tuple(jax.ShapeDtypeStruct((1, E), _F32) for _ in range(3)),
    )(eaT, r2r, dxr, dyr, dzr, w0r, w0e, b0c, w1c)


def kernel(x, h, v, edge_attr, edge_index,
           W_phi0, b_phi0, W_phi1, W_vel0, b_vel0, W_vel1, b_vel1):
    N = x.shape[0]
    E, D_EDGE = edge_attr.shape
    NL = W_phi0.shape[0]
    ALIGN = _NW * 128
    NP = ((N + ALIGN - 1) // ALIGN) * ALIGN
    EP = ((E + ALIGN - 1) // ALIGN) * ALIGN
    epad = jnp.full((EP - E,), N, jnp.int32)
    row = jnp.concatenate([edge_index[0], epad])
    col = jnp.concatenate([edge_index[1], epad])

    xp = jnp.zeros((3, NP), _F32).at[:, :N].set(x.T)
    vp = jnp.zeros((3, NP), _F32).at[:, :N].set(v.T)
    eaT = jnp.zeros((D_EDGE, EP), _F32).at[:, :E].set(edge_attr.T)
    CB = 128
    CH = (EP // _NW) // CB
    row3 = row.reshape(_NW, CH, CB)
    ones2 = jnp.ones((CH, CB), _F32)
    zerosN = jnp.zeros((NP,), _F32)

    w0r = W_phi0[:, 0:1, :].transpose(0, 2, 1)
    w0e = W_phi0[:, 1:, :].transpose(0, 2, 1)
    b0c = b_phi0[:, :, None]
    w1c = W_phi1
    wv0t = W_vel0.transpose(0, 2, 1)
    bv0c = b_vel0[:, :, None]
    bv1c = b_vel1[:, :, None]

    vs_all = _tc_vscale(vp, wv0t, bv0c, W_vel1, bv1c)

    gather_k = _make_sc_gather(NP, EP)
    scat_k = _make_sc_scatter(NP, CH, CB)
    upd_k = _make_sc_update(NP)

    for l in range(NL):
        dx, dy, dz, r2 = gather_k(xp[0], xp[1], xp[2], row, col)
        mx, my, mz = _tc_edge_mlp(
            eaT, r2.reshape(1, EP), dx.reshape(1, EP), dy.reshape(1, EP),
            dz.reshape(1, EP), w0r[l], w0e[l], b0c[l], w1c[l])
        parts = scat_k(
            row3, mx.reshape(_NW, CH, CB), my.reshape(_NW, CH, CB),
            mz.reshape(_NW, CH, CB), ones2, zerosN)
        xp = upd_k(xp, vp, vs_all[l], parts)

    xout = xp[:, :N].T
    return xout, h


# speedup vs baseline: 11.5335x; 1.1672x over previous
import functools

import jax
import jax.numpy as jnp
from jax import lax
from jax.experimental import pallas as pl
from jax.experimental.pallas import tpu as pltpu
from jax.experimental.pallas import tpu_sc as plsc

_F32 = jnp.float32
_NC = 2
_NS = 16
_NW = _NC * _NS
_L = 16
_GR = 8


def _mesh():
    return plsc.VectorSubcoreMesh(
        core_axis_name="c", subcore_axis_name="s",
        num_cores=_NC, num_subcores=_NS)


_SC_PARAMS = pltpu.CompilerParams(needs_layout_passes=False)


@functools.lru_cache(maxsize=None)
def _make_sc_gather(NP, E):
    EC = E // _NW

    @functools.partial(
        pl.kernel,
        mesh=_mesh(),
        compiler_params=_SC_PARAMS,
        out_type=tuple(jax.ShapeDtypeStruct((E,), _F32) for _ in range(4)),
        scratch_types=(
            [pltpu.VMEM((NP,), _F32)] * 3
            + [pltpu.VMEM((EC,), jnp.int32)] * 2
            + [pltpu.VMEM((EC,), _F32)] * 4
        ),
    )
    def k(x0, x1, x2, row, col, dx, dy, dz, r2,
          x0v, x1v, x2v, rowv, colv, dxv, dyv, dzv, r2v):
        wid = lax.axis_index("c") * _NS + lax.axis_index("s")
        base = wid * EC
        pltpu.sync_copy(x0, x0v)
        pltpu.sync_copy(x1, x1v)
        pltpu.sync_copy(x2, x2v)
        pltpu.sync_copy(row.at[pl.ds(base, EC)], rowv)
        pltpu.sync_copy(col.at[pl.ds(base, EC)], colv)

        @plsc.parallel_loop(0, EC, step=_L, unroll=8)
        def body(i):
            s = pl.ds(i, _L)
            rv = rowv[s]
            cv = colv[s]
            ax = plsc.load_gather(x0v, [rv]) - plsc.load_gather(x0v, [cv])
            ay = plsc.load_gather(x1v, [rv]) - plsc.load_gather(x1v, [cv])
            az = plsc.load_gather(x2v, [rv]) - plsc.load_gather(x2v, [cv])
            dxv[s] = ax
            dyv[s] = ay
            dzv[s] = az
            r2v[s] = ax * ax + ay * ay + az * az
        pltpu.sync_copy(dxv, dx.at[pl.ds(base, EC)])
        pltpu.sync_copy(dyv, dy.at[pl.ds(base, EC)])
        pltpu.sync_copy(dzv, dz.at[pl.ds(base, EC)])
        pltpu.sync_copy(r2v, r2.at[pl.ds(base, EC)])

    return k


@functools.lru_cache(maxsize=None)
def _make_sc_scatter(NP, EP):
    EC = EP // _NW
    SL = NP // _NS

    @functools.partial(
        pl.kernel,
        mesh=_mesh(),
        compiler_params=_SC_PARAMS,
        out_type=jax.ShapeDtypeStruct((_NC, 4, NP), _F32),
        scratch_types=(
            [pltpu.VMEM((EC,), jnp.int32)]
            + [pltpu.VMEM((EC,), _F32)] * 3
            + [pltpu.VMEM((NP,), _F32)] * 4
            + [pltpu.VMEM((4, SL), _F32)] * 2
            + [pltpu.VMEM_SHARED((_GR, 4, NP), _F32)]
        ),
    )
    def k(row, mx, my, mz, zerosN, out,
          rowv, mxv, myv, mzv, ax, ay, az, an, tmpv, sumv, stage):
        cid = lax.axis_index("c")
        sid = lax.axis_index("s")
        wid = cid * _NS + sid
        base = wid * EC
        pltpu.sync_copy(row.at[pl.ds(base, EC)], rowv)
        pltpu.sync_copy(mx.at[pl.ds(base, EC)], mxv)
        pltpu.sync_copy(my.at[pl.ds(base, EC)], myv)
        pltpu.sync_copy(mz.at[pl.ds(base, EC)], mzv)
        pltpu.sync_copy(zerosN, ax)
        pltpu.sync_copy(zerosN, ay)
        pltpu.sync_copy(zerosN, az)
        pltpu.sync_copy(zerosN, an)
        onev = jnp.full((_L,), 1.0, _F32)

        def body(i, carry):
            s = pl.ds(i * _L, _L)
            rv = rowv[s]
            plsc.addupdate_scatter(ax, [rv], mxv[s])
            plsc.addupdate_scatter(ay, [rv], myv[s])
            plsc.addupdate_scatter(az, [rv], mzv[s])
            plsc.addupdate_scatter(an, [rv], onev)
            return carry

        lax.fori_loop(0, EC // _L, body, 0, unroll=8)

        nbase = sid * SL

        @plsc.parallel_loop(0, SL, step=_L, unroll=4)
        def zero_sum(i):
            s = pl.ds(i, _L)
            zv = jnp.zeros((_L,), _F32)
            for c in range(4):
                sumv[c, s] = zv

        accs = (ax, ay, az, an)
        for r in range(_NS // _GR):
            @pl.when((sid >= r * _GR) & (sid < (r + 1) * _GR))
            def _stage():
                slot = sid - r * _GR
                for c in range(4):
                    pltpu.sync_copy(accs[c], stage.at[slot, c])

            plsc.subcore_barrier()
            for g in range(_GR):
                pltpu.sync_copy(stage.at[g, :, pl.ds(nbase, SL)], tmpv)

                def add_body(i2, c2):
                    s2 = pl.ds(i2 * _L, _L)
                    for c in range(4):
                        sumv[c, s2] = sumv[c, s2] + tmpv[c, s2]
                    return c2

                lax.fori_loop(0, SL // _L, add_body, 0, unroll=4)
            plsc.subcore_barrier()

        pltpu.sync_copy(sumv, out.at[cid, :, pl.ds(nbase, SL)])

    return k


@functools.lru_cache(maxsize=None)
def _make_sc_update(NP):
    CN = NP // _NW

    @functools.partial(
        pl.kernel,
        mesh=_mesh(),
        compiler_params=_SC_PARAMS,
        out_type=jax.ShapeDtypeStruct((3, NP), _F32),
        scratch_types=(
            [pltpu.VMEM((3, CN), _F32)] * 2
            + [pltpu.VMEM((CN,), _F32)]
            + [pltpu.VMEM((4, CN), _F32)] * 2
            + [pltpu.VMEM((3, CN), _F32)]
        ),
    )
    def k(xp, vp, vsl, parts, out, xv, vv, vsv, p0v, p1v, ov):
        wid = lax.axis_index("c") * _NS + lax.axis_index("s")
        base = wid * CN
        pltpu.sync_copy(xp.at[:, pl.ds(base, CN)], xv)
        pltpu.sync_copy(vp.at[:, pl.ds(base, CN)], vv)
        pltpu.sync_copy(vsl.at[pl.ds(base, CN)], vsv)
        pltpu.sync_copy(parts.at[0, :, pl.ds(base, CN)], p0v)
        pltpu.sync_copy(parts.at[1, :, pl.ds(base, CN)], p1v)

        @plsc.parallel_loop(0, CN, step=_L, unroll=4)
        def body(i):
            s = pl.ds(i, _L)
            cnt = p0v[3, s] + p1v[3, s]
            inv = 1.0 / jnp.maximum(cnt, 1.0)
            vs = vsv[s]
            for c in range(3):
                ov[c, s] = xv[c, s] + (p0v[c, s] + p1v[c, s]) * inv + vv[c, s] * vs
        pltpu.sync_copy(ov, out.at[:, pl.ds(base, CN)])

    return k


def _tc_vscale(vp, wv0t, bv0c, wv1, bv1c):
    NL, HID = wv0t.shape[0], wv0t.shape[1]
    NP = vp.shape[1]

    def body(vp_ref, w0_ref, b0_ref, w1_ref, b1_ref, o_ref):
        v0 = vp_ref[0:1, :]
        v1 = vp_ref[1:2, :]
        v2 = vp_ref[2:3, :]
        vn = jnp.sqrt(v0 * v0 + v1 * v1 + v2 * v2)
        hid = w0_ref[0] * vn + b0_ref[0]
        hid = hid * jax.nn.sigmoid(hid)
        vs = jnp.sum(hid * w1_ref[0], axis=0, keepdims=True) + b1_ref[0]
        o_ref[0] = vs

    return pl.pallas_call(
        body,
        grid=(NL,),
        in_specs=[
            pl.BlockSpec((3, NP), lambda l: (0, 0)),
            pl.BlockSpec((1, HID, 1), lambda l: (l, 0, 0)),
            pl.BlockSpec((1, HID, 1), lambda l: (l, 0, 0)),
            pl.BlockSpec((1, HID, 1), lambda l: (l, 0, 0)),
            pl.BlockSpec((1, 1, 1), lambda l: (l, 0, 0)),
        ],
        out_specs=pl.BlockSpec((1, 1, NP), lambda l: (l, 0, 0)),
        out_shape=jax.ShapeDtypeStruct((NL, 1, NP), _F32),
    )(vp, wv0t, bv0c, wv1, bv1c).reshape(NL, NP)


def _tc_edge_mlp(eaT, r2r, dxr, dyr, dzr, w0r, w0e, b0c, w1c):
    D_EDGE, E = eaT.shape
    HID = w0e.shape[0]
    BE = 2048
    G = E // BE

    def body(ea_ref, r2_ref, dx_ref, dy_ref, dz_ref,
             w0r_ref, w0e_ref, b0_ref, w1_ref, mx_ref, my_ref, mz_ref):
        rad = jnp.sqrt(r2_ref[...])
        hid = lax.dot_general(
            w0e_ref[...], ea_ref[...],
            (((1,), (0,)), ((), ())),
            preferred_element_type=_F32)
        hid = hid + w0r_ref[...] * rad + b0_ref[...]
        hid = hid * jax.nn.sigmoid(hid)
        t = jnp.tanh(jnp.sum(hid * w1_ref[...], axis=0, keepdims=True))
        mx_ref[...] = dx_ref[...] * t
        my_ref[...] = dy_ref[...] * t
        mz_ref[...] = dz_ref[...] * t

    return pl.pallas_call(
        body,
        grid=(G,),
        in_specs=[
            pl.BlockSpec((D_EDGE, BE), lambda j: (0, j)),
            pl.BlockSpec((1, BE), lambda j: (0, j)),
            pl.BlockSpec((1, BE), lambda j: (0, j)),
            pl.BlockSpec((1, BE), lambda j: (0, j)),
            pl.BlockSpec((1, BE), lambda j: (0, j)),
            pl.BlockSpec((HID, 1), lambda j: (0, 0)),
            pl.BlockSpec((HID, D_EDGE), lambda j: (0, 0)),
            pl.BlockSpec((HID, 1), lambda j: (0, 0)),
            pl.BlockSpec((HID, 1), lambda j: (0, 0)),
        ],
        out_specs=[pl.BlockSpec((1, BE), lambda j: (0, j))] * 3,
        out_shape=tuple(jax.ShapeDtypeStruct((1, E), _F32) for _ in range(3)),
    )(eaT, r2r, dxr, dyr, dzr, w0r, w0e, b0c, w1c)


def kernel(x, h, v, edge_attr, edge_index,
           W_phi0, b_phi0, W_phi1, W_vel0, b_vel0, W_vel1, b_vel1):
    N = x.shape[0]
    E, D_EDGE = edge_attr.shape
    NL = W_phi0.shape[0]
    ALIGN = _NW * 128
    NP = ((N + ALIGN - 1) // ALIGN) * ALIGN
    EP = ((E + ALIGN - 1) // ALIGN) * ALIGN
    epad = jnp.full((EP - E,), N, jnp.int32)
    row = jnp.concatenate([edge_index[0], epad])
    col = jnp.concatenate([edge_index[1], epad])

    xp = jnp.zeros((3, NP), _F32).at[:, :N].set(x.T)
    vp = jnp.zeros((3, NP), _F32).at[:, :N].set(v.T)
    eaT = jnp.zeros((D_EDGE, EP), _F32).at[:, :E].set(edge_attr.T)
    zerosN = jnp.zeros((NP,), _F32)

    w0r = W_phi0[:, 0:1, :].transpose(0, 2, 1)
    w0e = W_phi0[:, 1:, :].transpose(0, 2, 1)
    b0c = b_phi0[:, :, None]
    w1c = W_phi1
    wv0t = W_vel0.transpose(0, 2, 1)
    bv0c = b_vel0[:, :, None]
    bv1c = b_vel1[:, :, None]

    vs_all = _tc_vscale(vp, wv0t, bv0c, W_vel1, bv1c)

    gather_k = _make_sc_gather(NP, EP)
    scat_k = _make_sc_scatter(NP, EP)
    upd_k = _make_sc_update(NP)

    for l in range(NL):
        dx, dy, dz, r2 = gather_k(xp[0], xp[1], xp[2], row, col)
        mx, my, mz = _tc_edge_mlp(
            eaT, r2.reshape(1, EP), dx.reshape(1, EP), dy.reshape(1, EP),
            dz.reshape(1, EP), w0r[l], w0e[l], b0c[l], w1c[l])
        parts = scat_k(
            row, mx.reshape(EP), my.reshape(EP), mz.reshape(EP), zerosN)
        xp = upd_k(xp, vp, vs_all[l], parts)

    xout = xp[:, :N].T
    return xout, h


# speedup vs baseline: 11.6144x; 1.0070x over previous
import functools

import jax
import jax.numpy as jnp
from jax import lax
from jax.experimental import pallas as pl
from jax.experimental.pallas import tpu as pltpu
from jax.experimental.pallas import tpu_sc as plsc

_F32 = jnp.float32
_NC = 2
_NS = 16
_NW = _NC * _NS
_L = 16
_GR = 8


def _mesh():
    return plsc.VectorSubcoreMesh(
        core_axis_name="c", subcore_axis_name="s",
        num_cores=_NC, num_subcores=_NS)


_SC_PARAMS = pltpu.CompilerParams(needs_layout_passes=False)


@functools.lru_cache(maxsize=None)
def _make_sc_gather(NP, E):
    EC = E // _NW

    @functools.partial(
        pl.kernel,
        mesh=_mesh(),
        compiler_params=_SC_PARAMS,
        out_type=tuple(jax.ShapeDtypeStruct((E,), _F32) for _ in range(3)),
        scratch_types=(
            [pltpu.VMEM((NP,), _F32)] * 3
            + [pltpu.VMEM((EC,), jnp.int32)] * 2
            + [pltpu.VMEM((EC,), _F32)] * 3
        ),
    )
    def k(x0, x1, x2, row, col, dx, dy, dz,
          x0v, x1v, x2v, rowv, colv, dxv, dyv, dzv):
        wid = lax.axis_index("c") * _NS + lax.axis_index("s")
        base = wid * EC
        pltpu.sync_copy(x0, x0v)
        pltpu.sync_copy(x1, x1v)
        pltpu.sync_copy(x2, x2v)
        pltpu.sync_copy(row.at[pl.ds(base, EC)], rowv)
        pltpu.sync_copy(col.at[pl.ds(base, EC)], colv)

        @plsc.parallel_loop(0, EC, step=_L, unroll=8)
        def body(i):
            s = pl.ds(i, _L)
            rv = rowv[s]
            cv = colv[s]
            ax = plsc.load_gather(x0v, [rv]) - plsc.load_gather(x0v, [cv])
            ay = plsc.load_gather(x1v, [rv]) - plsc.load_gather(x1v, [cv])
            az = plsc.load_gather(x2v, [rv]) - plsc.load_gather(x2v, [cv])
            dxv[s] = ax
            dyv[s] = ay
            dzv[s] = az
        pltpu.sync_copy(dxv, dx.at[pl.ds(base, EC)])
        pltpu.sync_copy(dyv, dy.at[pl.ds(base, EC)])
        pltpu.sync_copy(dzv, dz.at[pl.ds(base, EC)])

    return k


@functools.lru_cache(maxsize=None)
def _make_sc_count(NP, EP):
    EC = EP // _NS
    SL = NP // _NS

    @functools.partial(
        pl.kernel,
        mesh=_mesh(),
        compiler_params=_SC_PARAMS,
        out_type=jax.ShapeDtypeStruct((NP,), _F32),
        scratch_types=(
            [pltpu.VMEM((EC,), jnp.int32)]
            + [pltpu.VMEM((NP,), _F32)]
            + [pltpu.VMEM((SL,), _F32)] * 2
            + [pltpu.VMEM_SHARED((_GR, 1, NP), _F32)]
        ),
    )
    def k(row, out, rowv, acc, tmps, sums, stage):
        cid = lax.axis_index("c")
        sid = lax.axis_index("s")
        base = sid * EC
        pltpu.sync_copy(row.at[pl.ds(base, EC)], rowv)

        @plsc.parallel_loop(0, NP, step=_L, unroll=4)
        def zacc(i):
            acc[pl.ds(i, _L)] = jnp.zeros((_L,), _F32)

        onev = jnp.full((_L,), 1.0, _F32)

        def body(i, carry):
            plsc.addupdate_scatter(acc, [rowv[pl.ds(i * _L, _L)]], onev)
            return carry

        lax.fori_loop(0, EC // _L, body, 0, unroll=8)

        nbase = sid * SL

        @plsc.parallel_loop(0, SL, step=_L, unroll=4)
        def zsum(i):
            sums[pl.ds(i, _L)] = jnp.zeros((_L,), _F32)

        for r in range(_NS // _GR):
            @pl.when((sid >= r * _GR) & (sid < (r + 1) * _GR))
            def _stage():
                pltpu.sync_copy(acc, stage.at[sid - r * _GR, 0])

            plsc.subcore_barrier()
            for g in range(_GR):
                pltpu.sync_copy(stage.at[g, 0, pl.ds(nbase, SL)], tmps)

                def add_body(i2, c2):
                    s2 = pl.ds(i2 * _L, _L)
                    sums[s2] = sums[s2] + tmps[s2]
                    return c2

                lax.fori_loop(0, SL // _L, add_body, 0, unroll=4)
            plsc.subcore_barrier()

        @plsc.parallel_loop(0, SL, step=_L, unroll=4)
        def inv(i):
            s = pl.ds(i, _L)
            sums[s] = 1.0 / jnp.maximum(sums[s], 1.0)

        @pl.when(cid == 0)
        def _w():
            pltpu.sync_copy(sums, out.at[pl.ds(nbase, SL)])

    return k


@functools.lru_cache(maxsize=None)
def _make_sc_scatter_update(NP, EP):
    EC = EP // _NS
    SL = NP // _NS
    GR = 4

    @functools.partial(
        pl.kernel,
        mesh=_mesh(),
        compiler_params=_SC_PARAMS,
        out_type=jax.ShapeDtypeStruct((3, NP), _F32),
        scratch_types=(
            [pltpu.VMEM((EC,), jnp.int32)]
            + [pltpu.VMEM((EC,), _F32)] * 2
            + [pltpu.VMEM((NP,), _F32)] * 3
            + [pltpu.VMEM((4, SL), _F32)] * 2
            + [pltpu.VMEM((3, SL), _F32)] * 3
            + [pltpu.VMEM((SL,), _F32)] * 2
            + [pltpu.SemaphoreType.DMA] * 2
            + [pltpu.VMEM_SHARED((GR, 4, NP), _F32)]
        ),
    )
    def k(row, mx, my, mz, cinv, xp, vp, vsl, out,
          rowv, mb0, mb1, ax, ay, az, tmpv, sumv, xv, vv, ov, vsv, civ,
          sem0, sem1, stage):
        cid = lax.axis_index("c")
        sid = lax.axis_index("s")
        base = sid * EC
        cp0 = pltpu.async_copy(mx.at[pl.ds(base, EC)], mb0, sem0)
        cp1 = pltpu.async_copy(my.at[pl.ds(base, EC)], mb1, sem1)
        pltpu.sync_copy(row.at[pl.ds(base, EC)], rowv)

        @plsc.parallel_loop(0, NP, step=_L, unroll=4)
        def zacc(i):
            s = pl.ds(i, _L)
            zv = jnp.zeros((_L,), _F32)
            ax[s] = zv
            ay[s] = zv
            az[s] = zv

        cp0.wait()

        def pass1(i, carry):
            s = pl.ds(i * _L, _L)
            plsc.addupdate_scatter(ax, [rowv[s]], mb0[s])
            return carry

        lax.fori_loop(0, EC // _L, pass1, 0, unroll=8)
        cp2 = pltpu.async_copy(mz.at[pl.ds(base, EC)], mb0, sem0)
        cp1.wait()

        def pass2(i, carry):
            s = pl.ds(i * _L, _L)
            plsc.addupdate_scatter(ay, [rowv[s]], mb1[s])
            return carry

        lax.fori_loop(0, EC // _L, pass2, 0, unroll=8)
        cp2.wait()

        def pass3(i, carry):
            s = pl.ds(i * _L, _L)
            plsc.addupdate_scatter(az, [rowv[s]], mb0[s])
            return carry

        lax.fori_loop(0, EC // _L, pass3, 0, unroll=8)

        nbase = sid * SL

        @plsc.parallel_loop(0, SL, step=_L, unroll=4)
        def zsum(i):
            s = pl.ds(i, _L)
            zv = jnp.zeros((_L,), _F32)
            for c in range(3):
                sumv[c, s] = zv

        accs = (ax, ay, az)
        for r in range(_NS // GR):
            @pl.when((sid >= r * GR) & (sid < (r + 1) * GR))
            def _stage():
                slot = sid - r * GR
                for c in range(3):
                    pltpu.sync_copy(accs[c], stage.at[slot, c])

            plsc.subcore_barrier()
            for g in range(GR):
                pltpu.sync_copy(stage.at[g, :, pl.ds(nbase, SL)], tmpv)

                def add_body(i2, c2):
                    s2 = pl.ds(i2 * _L, _L)
                    for c in range(3):
                        sumv[c, s2] = sumv[c, s2] + tmpv[c, s2]
                    return c2

                lax.fori_loop(0, SL // _L, add_body, 0, unroll=4)
            plsc.subcore_barrier()

        pltpu.sync_copy(xp.at[:, pl.ds(nbase, SL)], xv)
        pltpu.sync_copy(vp.at[:, pl.ds(nbase, SL)], vv)
        pltpu.sync_copy(vsl.at[pl.ds(nbase, SL)], vsv)
        pltpu.sync_copy(cinv.at[pl.ds(nbase, SL)], civ)

        @plsc.parallel_loop(0, SL, step=_L, unroll=4)
        def upd(i):
            s = pl.ds(i, _L)
            iv = civ[s]
            vs = vsv[s]
            for c in range(3):
                ov[c, s] = xv[c, s] + sumv[c, s] * iv + vv[c, s] * vs

        @pl.when(cid == 0)
        def _w():
            pltpu.sync_copy(ov, out.at[:, pl.ds(nbase, SL)])

    return k


def _tc_vscale(vp, wv0t, bv0c, wv1, bv1c):
    NL, HID = wv0t.shape[0], wv0t.shape[1]
    NP = vp.shape[1]

    def body(vp_ref, w0_ref, b0_ref, w1_ref, b1_ref, o_ref):
        v0 = vp_ref[0:1, :]
        v1 = vp_ref[1:2, :]
        v2 = vp_ref[2:3, :]
        vn = jnp.sqrt(v0 * v0 + v1 * v1 + v2 * v2)
        hid = w0_ref[0] * vn + b0_ref[0]
        hid = hid * jax.nn.sigmoid(hid)
        vs = jnp.sum(hid * w1_ref[0], axis=0, keepdims=True) + b1_ref[0]
        o_ref[0] = vs

    return pl.pallas_call(
        body,
        grid=(NL,),
        in_specs=[
            pl.BlockSpec((3, NP), lambda l: (0, 0)),
            pl.BlockSpec((1, HID, 1), lambda l: (l, 0, 0)),
            pl.BlockSpec((1, HID, 1), lambda l: (l, 0, 0)),
            pl.BlockSpec((1, HID, 1), lambda l: (l, 0, 0)),
            pl.BlockSpec((1, 1, 1), lambda l: (l, 0, 0)),
        ],
        out_specs=pl.BlockSpec((1, 1, NP), lambda l: (l, 0, 0)),
        out_shape=jax.ShapeDtypeStruct((NL, 1, NP), _F32),
    )(vp, wv0t, bv0c, wv1, bv1c).reshape(NL, NP)


def _tc_edge_mlp(eaT, dxr, dyr, dzr, w0r, w0e, b0c, w1c):
    D_EDGE, E = eaT.shape
    HID = w0e.shape[0]
    BE = 2048
    G = E // BE

    def body(ea_ref, dx_ref, dy_ref, dz_ref,
             w0r_ref, w0e_ref, b0_ref, w1_ref, mx_ref, my_ref, mz_ref):
        dxb = dx_ref[...]
        dyb = dy_ref[...]
        dzb = dz_ref[...]
        rad = jnp.sqrt(dxb * dxb + dyb * dyb + dzb * dzb)
        hid = lax.dot_general(
            w0e_ref[...], ea_ref[...],
            (((1,), (0,)), ((), ())),
            preferred_element_type=_F32)
        hid = hid + w0r_ref[...] * rad + b0_ref[...]
        hid = hid * jax.nn.sigmoid(hid)
        t = jnp.tanh(jnp.sum(hid * w1_ref[...], axis=0, keepdims=True))
        mx_ref[...] = dxb * t
        my_ref[...] = dyb * t
        mz_ref[...] = dzb * t

    return pl.pallas_call(
        body,
        grid=(G,),
        in_specs=[
            pl.BlockSpec((D_EDGE, BE), lambda j: (0, j)),
            pl.BlockSpec((1, BE), lambda j: (0, j)),
            pl.BlockSpec((1, BE), lambda j: (0, j)),
            pl.BlockSpec((1, BE), lambda j: (0, j)),
            pl.BlockSpec((HID, 1), lambda j: (0, 0)),
            pl.BlockSpec((HID, D_EDGE), lambda j: (0, 0)),
            pl.BlockSpec((HID, 1), lambda j: (0, 0)),
            pl.BlockSpec((HID, 1), lambda j: (0, 0)),
        ],
        out_specs=[pl.BlockSpec((1, BE), lambda j: (0, j))] * 3,
        out_shape=tuple(jax.ShapeDtypeStruct((1, E), _F32) for _ in range(3)),
    )(eaT, dxr, dyr, dzr, w0r, w0e, b0c, w1c)


def kernel(x, h, v, edge_attr, edge_index,
           W_phi0, b_phi0, W_phi1, W_vel0, b_vel0, W_vel1, b_vel1):
    N = x.shape[0]
    E, D_EDGE = edge_attr.shape
    NL = W_phi0.shape[0]
    ALIGN = _NW * 128
    NP = ((N + ALIGN - 1) // ALIGN) * ALIGN
    EP = ((E + ALIGN - 1) // ALIGN) * ALIGN
    epad = jnp.full((EP - E,), N, jnp.int32)
    row = jnp.concatenate([edge_index[0], epad])
    col = jnp.concatenate([edge_index[1], epad])

    xp = jnp.zeros((3, NP), _F32).at[:, :N].set(x.T)
    vp = jnp.zeros((3, NP), _F32).at[:, :N].set(v.T)
    eaT = jnp.zeros((D_EDGE, EP), _F32).at[:, :E].set(edge_attr.T)

    w0r = W_phi0[:, 0:1, :].transpose(0, 2, 1)
    w0e = W_phi0[:, 1:, :].transpose(0, 2, 1)
    b0c = b_phi0[:, :, None]
    w1c = W_phi1
    wv0t = W_vel0.transpose(0, 2, 1)
    bv0c = b_vel0[:, :, None]
    bv1c = b_vel1[:, :, None]

    vs_all = _tc_vscale(vp, wv0t, bv0c, W_vel1, bv1c)

    gather_k = _make_sc_gather(NP, EP)
    count_k = _make_sc_count(NP, EP)
    scatupd_k = _make_sc_scatter_update(NP, EP)

    cnt_inv = count_k(row)
    for l in range(NL):
        dx, dy, dz = gather_k(xp[0], xp[1], xp[2], row, col)
        mx, my, mz = _tc_edge_mlp(
            eaT, dx.reshape(1, EP), dy.reshape(1, EP),
            dz.reshape(1, EP), w0r[l], w0e[l], b0c[l], w1c[l])
        xp = scatupd_k(row, mx.reshape(EP), my.reshape(EP), mz.reshape(EP),
                       cnt_inv, xp, vp, vs_all[l])

    xout = xp[:, :N].T
    return xout, h


# speedup vs baseline: 11.8831x; 1.0231x over previous
import functools

import jax
import jax.numpy as jnp
from jax import lax
from jax.experimental import pallas as pl
from jax.experimental.pallas import tpu as pltpu
from jax.experimental.pallas import tpu_sc as plsc

_F32 = jnp.float32
_NC = 2
_NS = 16
_NW = _NC * _NS
_L = 16
_GR = 8


def _mesh():
    return plsc.VectorSubcoreMesh(
        core_axis_name="c", subcore_axis_name="s",
        num_cores=_NC, num_subcores=_NS)


_SC_PARAMS = pltpu.CompilerParams(needs_layout_passes=False)


@functools.lru_cache(maxsize=None)
def _make_sc_gather(NP, E):
    EC = E // _NW

    @functools.partial(
        pl.kernel,
        mesh=_mesh(),
        compiler_params=_SC_PARAMS,
        out_type=tuple(jax.ShapeDtypeStruct((E,), _F32) for _ in range(3)),
        scratch_types=(
            [pltpu.VMEM((NP,), _F32)] * 3
            + [pltpu.VMEM((EC,), jnp.int32)]
            + [pltpu.VMEM((EC,), _F32)] * 3
            + [pltpu.SemaphoreType.DMA] * 2
        ),
    )
    def k(rc, x0, x1, x2, dx, dy, dz,
          x0v, x1v, x2v, rcv, dxv, dyv, dzv, semi, semo):
        wid = lax.axis_index("c") * _NS + lax.axis_index("s")
        base = wid * EC
        cps = [
            pltpu.async_copy(rc.at[pl.ds(base, EC)], rcv, semi),
            pltpu.async_copy(x0, x0v, semi),
            pltpu.async_copy(x1, x1v, semi),
            pltpu.async_copy(x2, x2v, semi),
        ]
        for cp in cps:
            cp.wait()

        @plsc.parallel_loop(0, EC, step=_L, unroll=8)
        def body(i):
            s = pl.ds(i, _L)
            rcw = rcv[s]
            rv = rcw & 0x3FFF
            cv = rcw >> 14
            ax = plsc.load_gather(x0v, [rv]) - plsc.load_gather(x0v, [cv])
            ay = plsc.load_gather(x1v, [rv]) - plsc.load_gather(x1v, [cv])
            az = plsc.load_gather(x2v, [rv]) - plsc.load_gather(x2v, [cv])
            dxv[s] = ax
            dyv[s] = ay
            dzv[s] = az

        cpo = [
            pltpu.async_copy(dxv, dx.at[pl.ds(base, EC)], semo),
            pltpu.async_copy(dyv, dy.at[pl.ds(base, EC)], semo),
            pltpu.async_copy(dzv, dz.at[pl.ds(base, EC)], semo),
        ]
        for cp in cpo:
            cp.wait()

    return k


@functools.lru_cache(maxsize=None)
def _make_sc_count(NP, EP):
    EC = EP // _NS
    SL = NP // _NS

    @functools.partial(
        pl.kernel,
        mesh=_mesh(),
        compiler_params=_SC_PARAMS,
        out_type=jax.ShapeDtypeStruct((NP,), _F32),
        scratch_types=(
            [pltpu.VMEM((EC,), jnp.int32)]
            + [pltpu.VMEM((NP,), _F32)]
            + [pltpu.VMEM((SL,), _F32)] * 2
            + [pltpu.VMEM_SHARED((_GR, 1, NP), _F32)]
        ),
    )
    def k(row, out, rowv, acc, tmps, sums, stage):
        cid = lax.axis_index("c")
        sid = lax.axis_index("s")
        base = sid * EC
        pltpu.sync_copy(row.at[pl.ds(base, EC)], rowv)

        @plsc.parallel_loop(0, NP, step=_L, unroll=4)
        def zacc(i):
            acc[pl.ds(i, _L)] = jnp.zeros((_L,), _F32)

        onev = jnp.full((_L,), 1.0, _F32)

        def body(i, carry):
            plsc.addupdate_scatter(acc, [rowv[pl.ds(i * _L, _L)]], onev)
            return carry

        lax.fori_loop(0, EC // _L, body, 0, unroll=8)

        nbase = sid * SL

        @plsc.parallel_loop(0, SL, step=_L, unroll=4)
        def zsum(i):
            sums[pl.ds(i, _L)] = jnp.zeros((_L,), _F32)

        for r in range(_NS // _GR):
            @pl.when((sid >= r * _GR) & (sid < (r + 1) * _GR))
            def _stage():
                pltpu.sync_copy(acc, stage.at[sid - r * _GR, 0])

            plsc.subcore_barrier()
            for g in range(_GR):
                pltpu.sync_copy(stage.at[g, 0, pl.ds(nbase, SL)], tmps)

                def add_body(i2, c2):
                    s2 = pl.ds(i2 * _L, _L)
                    sums[s2] = sums[s2] + tmps[s2]
                    return c2

                lax.fori_loop(0, SL // _L, add_body, 0, unroll=4)
            plsc.subcore_barrier()

        @plsc.parallel_loop(0, SL, step=_L, unroll=4)
        def inv(i):
            s = pl.ds(i, _L)
            sums[s] = 1.0 / jnp.maximum(sums[s], 1.0)

        @pl.when(cid == 0)
        def _w():
            pltpu.sync_copy(sums, out.at[pl.ds(nbase, SL)])

    return k


@functools.lru_cache(maxsize=None)
def _make_sc_scatter_update(NP, EP):
    EC = EP // _NS
    EH = EC // 2
    SL = NP // _NS
    GR = 4

    @functools.partial(
        pl.kernel,
        mesh=_mesh(),
        compiler_params=_SC_PARAMS,
        out_type=jax.ShapeDtypeStruct((3, NP), _F32),
        scratch_types=(
            [pltpu.VMEM((EC,), jnp.int32)]
            + [pltpu.VMEM((EC,), _F32)] * 2
            + [pltpu.VMEM((NP,), _F32)] * 3
            + [pltpu.VMEM((4, SL), _F32)] * 2
            + [pltpu.VMEM((3, SL), _F32)] * 3
            + [pltpu.VMEM((SL,), _F32)] * 2
            + [pltpu.SemaphoreType.DMA] * 4
            + [pltpu.VMEM_SHARED((GR, 4, NP), _F32)]
        ),
    )
    def k(row, mx, my, mz, cinv, xp, vp, vsl, out,
          rowv, mb0, mb1, ax, ay, az, tmpv, sumv, xv, vv, ov, vsv, civ,
          semr, sem0, sem1, sems, stage):
        cid = lax.axis_index("c")
        sid = lax.axis_index("s")
        base = sid * EC
        nbase = sid * SL
        cpr = [
            pltpu.async_copy(row.at[pl.ds(base, EH)], rowv.at[pl.ds(0, EH)], semr),
            pltpu.async_copy(row.at[pl.ds(base + EH, EH)], rowv.at[pl.ds(EH, EH)], semr),
        ]
        cp0 = [
            pltpu.async_copy(mx.at[pl.ds(base, EH)], mb0.at[pl.ds(0, EH)], sem0),
            pltpu.async_copy(mx.at[pl.ds(base + EH, EH)], mb0.at[pl.ds(EH, EH)], sem0),
        ]
        cp1 = [
            pltpu.async_copy(my.at[pl.ds(base, EH)], mb1.at[pl.ds(0, EH)], sem1),
            pltpu.async_copy(my.at[pl.ds(base + EH, EH)], mb1.at[pl.ds(EH, EH)], sem1),
        ]
        cps = [
            pltpu.async_copy(xp.at[:, pl.ds(nbase, SL)], xv, sems),
            pltpu.async_copy(vp.at[:, pl.ds(nbase, SL)], vv, sems),
            pltpu.async_copy(vsl.at[pl.ds(nbase, SL)], vsv, sems),
            pltpu.async_copy(cinv.at[pl.ds(nbase, SL)], civ, sems),
        ]

        @plsc.parallel_loop(0, NP, step=_L, unroll=4)
        def zacc(i):
            s = pl.ds(i, _L)
            zv = jnp.zeros((_L,), _F32)
            ax[s] = zv
            ay[s] = zv
            az[s] = zv

        for cp in cpr:
            cp.wait()
        for cp in cp0:
            cp.wait()

        def pass1(i, carry):
            s = pl.ds(i * _L, _L)
            plsc.addupdate_scatter(ax, [rowv[s]], mb0[s])
            return carry

        lax.fori_loop(0, EC // _L, pass1, 0, unroll=8)
        cp2 = [
            pltpu.async_copy(mz.at[pl.ds(base, EH)], mb0.at[pl.ds(0, EH)], sem0),
            pltpu.async_copy(mz.at[pl.ds(base + EH, EH)], mb0.at[pl.ds(EH, EH)], sem0),
        ]
        for cp in cp1:
            cp.wait()

        def pass2(i, carry):
            s = pl.ds(i * _L, _L)
            plsc.addupdate_scatter(ay, [rowv[s]], mb1[s])
            return carry

        lax.fori_loop(0, EC // _L, pass2, 0, unroll=8)
        for cp in cp2:
            cp.wait()

        def pass3(i, carry):
            s = pl.ds(i * _L, _L)
            plsc.addupdate_scatter(az, [rowv[s]], mb0[s])
            return carry

        lax.fori_loop(0, EC // _L, pass3, 0, unroll=8)

        @plsc.parallel_loop(0, SL, step=_L, unroll=4)
        def zsum(i):
            s = pl.ds(i, _L)
            zv = jnp.zeros((_L,), _F32)
            for c in range(3):
                sumv[c, s] = zv

        accs = (ax, ay, az)
        for r in range(_NS // GR):
            @pl.when((sid >= r * GR) & (sid < (r + 1) * GR))
            def _stage():
                slot = sid - r * GR
                for c in range(3):
                    pltpu.sync_copy(accs[c], stage.at[slot, c])

            plsc.subcore_barrier()
            for g in range(GR):
                pltpu.sync_copy(stage.at[g, :, pl.ds(nbase, SL)], tmpv)

                def add_body(i2, c2):
                    s2 = pl.ds(i2 * _L, _L)
                    for c in range(3):
                        sumv[c, s2] = sumv[c, s2] + tmpv[c, s2]
                    return c2

                lax.fori_loop(0, SL // _L, add_body, 0, unroll=4)
            plsc.subcore_barrier()

        for cp in cps:
            cp.wait()

        @plsc.parallel_loop(0, SL, step=_L, unroll=4)
        def upd(i):
            s = pl.ds(i, _L)
            iv = civ[s]
            vs = vsv[s]
            for c in range(3):
                ov[c, s] = xv[c, s] + sumv[c, s] * iv + vv[c, s] * vs

        @pl.when(cid == 0)
        def _w():
            pltpu.sync_copy(ov, out.at[:, pl.ds(nbase, SL)])

    return k


def _tc_vscale(vp, wv0t, bv0c, wv1, bv1c):
    NL, HID = wv0t.shape[0], wv0t.shape[1]
    NP = vp.shape[1]

    def body(vp_ref, w0_ref, b0_ref, w1_ref, b1_ref, o_ref):
        v0 = vp_ref[0:1, :]
        v1 = vp_ref[1:2, :]
        v2 = vp_ref[2:3, :]
        vn = jnp.sqrt(v0 * v0 + v1 * v1 + v2 * v2)
        hid = w0_ref[0] * vn + b0_ref[0]
        hid = hid * jax.nn.sigmoid(hid)
        vs = jnp.sum(hid * w1_ref[0], axis=0, keepdims=True) + b1_ref[0]
        o_ref[0] = vs

    return pl.pallas_call(
        body,
        grid=(NL,),
        in_specs=[
            pl.BlockSpec((3, NP), lambda l: (0, 0)),
            pl.BlockSpec((1, HID, 1), lambda l: (l, 0, 0)),
            pl.BlockSpec((1, HID, 1), lambda l: (l, 0, 0)),
            pl.BlockSpec((1, HID, 1), lambda l: (l, 0, 0)),
            pl.BlockSpec((1, 1, 1), lambda l: (l, 0, 0)),
        ],
        out_specs=pl.BlockSpec((1, 1, NP), lambda l: (l, 0, 0)),
        out_shape=jax.ShapeDtypeStruct((NL, 1, NP), _F32),
    )(vp, wv0t, bv0c, wv1, bv1c).reshape(NL, NP)


def _tc_edge_mlp(eaT, dxr, dyr, dzr, w0r, w0e, b0c, w1c):
    D_EDGE, E = eaT.shape
    HID = w0e.shape[0]
    BE = 2048
    G = E // BE

    def body(ea_ref, dx_ref, dy_ref, dz_ref,
             w0r_ref, w0e_ref, b0_ref, w1_ref, mx_ref, my_ref, mz_ref):
        dxb = dx_ref[...]
        dyb = dy_ref[...]
        dzb = dz_ref[...]
        rad = jnp.sqrt(dxb * dxb + dyb * dyb + dzb * dzb)
        hid = lax.dot_general(
            w0e_ref[...], ea_ref[...],
            (((1,), (0,)), ((), ())),
            preferred_element_type=_F32)
        hid = hid + w0r_ref[...] * rad + b0_ref[...]
        hid = hid * jax.nn.sigmoid(hid)
        t = jnp.tanh(jnp.sum(hid * w1_ref[...], axis=0, keepdims=True))
        mx_ref[...] = dxb * t
        my_ref[...] = dyb * t
        mz_ref[...] = dzb * t

    return pl.pallas_call(
        body,
        grid=(G,),
        in_specs=[
            pl.BlockSpec((D_EDGE, BE), lambda j: (0, j)),
            pl.BlockSpec((1, BE), lambda j: (0, j)),
            pl.BlockSpec((1, BE), lambda j: (0, j)),
            pl.BlockSpec((1, BE), lambda j: (0, j)),
            pl.BlockSpec((HID, 1), lambda j: (0, 0)),
            pl.BlockSpec((HID, D_EDGE), lambda j: (0, 0)),
            pl.BlockSpec((HID, 1), lambda j: (0, 0)),
            pl.BlockSpec((HID, 1), lambda j: (0, 0)),
        ],
        out_specs=[pl.BlockSpec((1, BE), lambda j: (0, j))] * 3,
        out_shape=tuple(jax.ShapeDtypeStruct((1, E), _F32) for _ in range(3)),
    )(eaT, dxr, dyr, dzr, w0r, w0e, b0c, w1c)


def kernel(x, h, v, edge_attr, edge_index,
           W_phi0, b_phi0, W_phi1, W_vel0, b_vel0, W_vel1, b_vel1):
    N = x.shape[0]
    E, D_EDGE = edge_attr.shape
    NL = W_phi0.shape[0]
    ALIGN = _NW * 128
    NP = ((N + ALIGN - 1) // ALIGN) * ALIGN
    EP = ((E + ALIGN - 1) // ALIGN) * ALIGN
    epad = jnp.full((EP - E,), N, jnp.int32)
    row = jnp.concatenate([edge_index[0], epad])
    col = jnp.concatenate([edge_index[1], epad])
    rc = jnp.bitwise_or(row, jnp.left_shift(col, 14))

    xp = jnp.zeros((3, NP), _F32).at[:, :N].set(x.T)
    vp = jnp.zeros((3, NP), _F32).at[:, :N].set(v.T)
    eaT = jnp.zeros((D_EDGE, EP), _F32).at[:, :E].set(edge_attr.T)

    w0r = W_phi0[:, 0:1, :].transpose(0, 2, 1)
    w0e = W_phi0[:, 1:, :].transpose(0, 2, 1)
    b0c = b_phi0[:, :, None]
    w1c = W_phi1
    wv0t = W_vel0.transpose(0, 2, 1)
    bv0c = b_vel0[:, :, None]
    bv1c = b_vel1[:, :, None]

    vs_all = _tc_vscale(vp, wv0t, bv0c, W_vel1, bv1c)

    gather_k = _make_sc_gather(NP, EP)
    count_k = _make_sc_count(NP, EP)
    scatupd_k = _make_sc_scatter_update(NP, EP)

    cnt_inv = count_k(row)
    for l in range(NL):
        dx, dy, dz = gather_k(rc, xp[0], xp[1], xp[2])
        mx, my, mz = _tc_edge_mlp(
            eaT, dx.reshape(1, EP), dy.reshape(1, EP),
            dz.reshape(1, EP), w0r[l], w0e[l], b0c[l], w1c[l])
        xp = scatupd_k(row, mx.reshape(EP), my.reshape(EP), mz.reshape(EP),
                       cnt_inv, xp, vp, vs_all[l])

    xout = xp[:, :N].T
    return xout, h


# speedup vs baseline: 11.9599x; 1.0065x over previous
import functools

import jax
import jax.numpy as jnp
from jax import lax
from jax.experimental import pallas as pl
from jax.experimental.pallas import tpu as pltpu
from jax.experimental.pallas import tpu_sc as plsc

_F32 = jnp.float32
_NC = 2
_NS = 16
_NW = _NC * _NS
_L = 16
_GR = 8


def _mesh():
    return plsc.VectorSubcoreMesh(
        core_axis_name="c", subcore_axis_name="s",
        num_cores=_NC, num_subcores=_NS)


_SC_PARAMS = pltpu.CompilerParams(needs_layout_passes=False)


@functools.lru_cache(maxsize=None)
def _make_sc_gather(NP, E):
    EC = E // _NW

    @functools.partial(
        pl.kernel,
        mesh=_mesh(),
        compiler_params=_SC_PARAMS,
        out_type=jax.ShapeDtypeStruct((E,), _F32),
        scratch_types=(
            [pltpu.VMEM((NP,), _F32)] * 3
            + [pltpu.VMEM((EC,), jnp.int32)]
            + [pltpu.VMEM((EC,), _F32)]
            + [pltpu.SemaphoreType.DMA]
        ),
    )
    def k(rc, x0, x1, x2, r2,
          x0v, x1v, x2v, rcv, r2v, semi):
        wid = lax.axis_index("c") * _NS + lax.axis_index("s")
        base = wid * EC
        cps = [
            pltpu.async_copy(rc.at[pl.ds(base, EC)], rcv, semi),
            pltpu.async_copy(x0, x0v, semi),
            pltpu.async_copy(x1, x1v, semi),
            pltpu.async_copy(x2, x2v, semi),
        ]
        for cp in cps:
            cp.wait()

        @plsc.parallel_loop(0, EC, step=_L, unroll=8)
        def body(i):
            s = pl.ds(i, _L)
            rcw = rcv[s]
            rv = rcw & 0x3FFF
            cv = rcw >> 14
            ax = plsc.load_gather(x0v, [rv]) - plsc.load_gather(x0v, [cv])
            ay = plsc.load_gather(x1v, [rv]) - plsc.load_gather(x1v, [cv])
            az = plsc.load_gather(x2v, [rv]) - plsc.load_gather(x2v, [cv])
            r2v[s] = ax * ax + ay * ay + az * az

        pltpu.sync_copy(r2v, r2.at[pl.ds(base, EC)])

    return k


@functools.lru_cache(maxsize=None)
def _make_sc_count(NP, EP):
    EC = EP // _NS
    SL = NP // _NS

    @functools.partial(
        pl.kernel,
        mesh=_mesh(),
        compiler_params=_SC_PARAMS,
        out_type=jax.ShapeDtypeStruct((NP,), _F32),
        scratch_types=(
            [pltpu.VMEM((EC,), jnp.int32)]
            + [pltpu.VMEM((NP,), _F32)]
            + [pltpu.VMEM((SL,), _F32)] * 2
            + [pltpu.VMEM_SHARED((_GR, 1, NP), _F32)]
        ),
    )
    def k(row, out, rowv, acc, tmps, sums, stage):
        cid = lax.axis_index("c")
        sid = lax.axis_index("s")
        base = sid * EC
        pltpu.sync_copy(row.at[pl.ds(base, EC)], rowv)

        @plsc.parallel_loop(0, NP, step=_L, unroll=4)
        def zacc(i):
            acc[pl.ds(i, _L)] = jnp.zeros((_L,), _F32)

        onev = jnp.full((_L,), 1.0, _F32)

        def body(i, carry):
            plsc.addupdate_scatter(acc, [rowv[pl.ds(i * _L, _L)]], onev)
            return carry

        lax.fori_loop(0, EC // _L, body, 0, unroll=8)

        nbase = sid * SL

        @plsc.parallel_loop(0, SL, step=_L, unroll=4)
        def zsum(i):
            sums[pl.ds(i, _L)] = jnp.zeros((_L,), _F32)

        for r in range(_NS // _GR):
            @pl.when((sid >= r * _GR) & (sid < (r + 1) * _GR))
            def _stage():
                pltpu.sync_copy(acc, stage.at[sid - r * _GR, 0])

            plsc.subcore_barrier()
            for g in range(_GR):
                pltpu.sync_copy(stage.at[g, 0, pl.ds(nbase, SL)], tmps)

                def add_body(i2, c2):
                    s2 = pl.ds(i2 * _L, _L)
                    sums[s2] = sums[s2] + tmps[s2]
                    return c2

                lax.fori_loop(0, SL // _L, add_body, 0, unroll=4)
            plsc.subcore_barrier()

        @plsc.parallel_loop(0, SL, step=_L, unroll=4)
        def inv(i):
            s = pl.ds(i, _L)
            sums[s] = 1.0 / jnp.maximum(sums[s], 1.0)

        @pl.when(cid == 0)
        def _w():
            pltpu.sync_copy(sums, out.at[pl.ds(nbase, SL)])

    return k


@functools.lru_cache(maxsize=None)
def _make_sc_scatter_update(NP, EP):
    EC = EP // _NS
    EH = EC // 2
    SL = NP // _NS
    GR = 4

    @functools.partial(
        pl.kernel,
        mesh=_mesh(),
        compiler_params=_SC_PARAMS,
        out_type=jax.ShapeDtypeStruct((3, NP), _F32),
        scratch_types=(
            [pltpu.VMEM((EC,), jnp.int32)]
            + [pltpu.VMEM((EC,), _F32)]
            + [pltpu.VMEM((NP,), _F32)] * 6
            + [pltpu.VMEM((4, SL), _F32)] * 2
            + [pltpu.VMEM((3, SL), _F32)] * 2
            + [pltpu.VMEM((SL,), _F32)] * 2
            + [pltpu.SemaphoreType.DMA] * 2
            + [pltpu.VMEM_SHARED((GR, 4, NP), _F32)]
        ),
    )
    def k(rc, t, cinv, x0, x1, x2, vp, vsl, out,
          rcv, tv, x0v, x1v, x2v, ax, ay, az, tmpv, sumv, vv, ov, vsv, civ,
          semi, sems, stage):
        cid = lax.axis_index("c")
        sid = lax.axis_index("s")
        base = sid * EC
        nbase = sid * SL
        cpi = [
            pltpu.async_copy(rc.at[pl.ds(base, EH)], rcv.at[pl.ds(0, EH)], semi),
            pltpu.async_copy(rc.at[pl.ds(base + EH, EH)], rcv.at[pl.ds(EH, EH)], semi),
            pltpu.async_copy(t.at[pl.ds(base, EH)], tv.at[pl.ds(0, EH)], semi),
            pltpu.async_copy(t.at[pl.ds(base + EH, EH)], tv.at[pl.ds(EH, EH)], semi),
            pltpu.async_copy(x0, x0v, semi),
            pltpu.async_copy(x1, x1v, semi),
            pltpu.async_copy(x2, x2v, semi),
        ]
        cps = [
            pltpu.async_copy(vp.at[:, pl.ds(nbase, SL)], vv, sems),
            pltpu.async_copy(vsl.at[pl.ds(nbase, SL)], vsv, sems),
            pltpu.async_copy(cinv.at[pl.ds(nbase, SL)], civ, sems),
        ]

        @plsc.parallel_loop(0, NP, step=_L, unroll=4)
        def zacc(i):
            s = pl.ds(i, _L)
            zv = jnp.zeros((_L,), _F32)
            ax[s] = zv
            ay[s] = zv
            az[s] = zv

        for cp in cpi:
            cp.wait()

        def body(i, carry):
            s = pl.ds(i * _L, _L)
            rcw = rcv[s]
            rv = rcw & 0x3FFF
            cv = rcw >> 14
            tw = tv[s]
            mxw = (plsc.load_gather(x0v, [rv]) - plsc.load_gather(x0v, [cv])) * tw
            myw = (plsc.load_gather(x1v, [rv]) - plsc.load_gather(x1v, [cv])) * tw
            mzw = (plsc.load_gather(x2v, [rv]) - plsc.load_gather(x2v, [cv])) * tw
            plsc.addupdate_scatter(ax, [rv], mxw)
            plsc.addupdate_scatter(ay, [rv], myw)
            plsc.addupdate_scatter(az, [rv], mzw)
            return carry

        lax.fori_loop(0, EC // _L, body, 0, unroll=8)

        @plsc.parallel_loop(0, SL, step=_L, unroll=4)
        def zsum(i):
            s = pl.ds(i, _L)
            zv = jnp.zeros((_L,), _F32)
            for c in range(3):
                sumv[c, s] = zv

        accs = (ax, ay, az)
        for r in range(_NS // GR):
            @pl.when((sid >= r * GR) & (sid < (r + 1) * GR))
            def _stage():
                slot = sid - r * GR
                for c in range(3):
                    pltpu.sync_copy(accs[c], stage.at[slot, c])

            plsc.subcore_barrier()
            for g in range(GR):
                pltpu.sync_copy(stage.at[g, :, pl.ds(nbase, SL)], tmpv)

                def add_body(i2, c2):
                    s2 = pl.ds(i2 * _L, _L)
                    for c in range(3):
                        sumv[c, s2] = sumv[c, s2] + tmpv[c, s2]
                    return c2

                lax.fori_loop(0, SL // _L, add_body, 0, unroll=4)
            plsc.subcore_barrier()

        for cp in cps:
            cp.wait()

        xslices = (x0v, x1v, x2v)

        @plsc.parallel_loop(0, SL, step=_L, unroll=4)
        def upd(i):
            s = pl.ds(i, _L)
            sx = pl.ds(nbase + i, _L)
            iv = civ[s]
            vs = vsv[s]
            for c in range(3):
                ov[c, s] = xslices[c][sx] + sumv[c, s] * iv + vv[c, s] * vs

        @pl.when(cid == 0)
        def _w():
            pltpu.sync_copy(ov, out.at[:, pl.ds(nbase, SL)])

    return k


def _tc_vscale(vp, wv0t, bv0c, wv1, bv1c):
    NL, HID = wv0t.shape[0], wv0t.shape[1]
    NP = vp.shape[1]

    def body(vp_ref, w0_ref, b0_ref, w1_ref, b1_ref, o_ref):
        v0 = vp_ref[0:1, :]
        v1 = vp_ref[1:2, :]
        v2 = vp_ref[2:3, :]
        vn = jnp.sqrt(v0 * v0 + v1 * v1 + v2 * v2)
        hid = w0_ref[0] * vn + b0_ref[0]
        hid = hid * jax.nn.sigmoid(hid)
        vs = jnp.sum(hid * w1_ref[0], axis=0, keepdims=True) + b1_ref[0]
        o_ref[0] = vs

    return pl.pallas_call(
        body,
        grid=(NL,),
        in_specs=[
            pl.BlockSpec((3, NP), lambda l: (0, 0)),
            pl.BlockSpec((1, HID, 1), lambda l: (l, 0, 0)),
            pl.BlockSpec((1, HID, 1), lambda l: (l, 0, 0)),
            pl.BlockSpec((1, HID, 1), lambda l: (l, 0, 0)),
            pl.BlockSpec((1, 1, 1), lambda l: (l, 0, 0)),
        ],
        out_specs=pl.BlockSpec((1, 1, NP), lambda l: (l, 0, 0)),
        out_shape=jax.ShapeDtypeStruct((NL, 1, NP), _F32),
    )(vp, wv0t, bv0c, wv1, bv1c).reshape(NL, NP)


def _tc_edge_mlp(eaT, r2r, w0r, w0e, b0c, w1c):
    D_EDGE, E = eaT.shape
    HID = w0e.shape[0]
    BE = 2048
    G = E // BE

    def body(ea_ref, r2_ref, w0r_ref, w0e_ref, b0_ref, w1_ref, t_ref):
        rad = jnp.sqrt(r2_ref[...])
        hid = lax.dot_general(
            w0e_ref[...], ea_ref[...],
            (((1,), (0,)), ((), ())),
            preferred_element_type=_F32)
        hid = hid + w0r_ref[...] * rad + b0_ref[...]
        hid = hid * jax.nn.sigmoid(hid)
        t_ref[...] = jnp.tanh(
            jnp.sum(hid * w1_ref[...], axis=0, keepdims=True))

    return pl.pallas_call(
        body,
        grid=(G,),
        in_specs=[
            pl.BlockSpec((D_EDGE, BE), lambda j: (0, j)),
            pl.BlockSpec((1, BE), lambda j: (0, j)),
            pl.BlockSpec((HID, 1), lambda j: (0, 0)),
            pl.BlockSpec((HID, D_EDGE), lambda j: (0, 0)),
            pl.BlockSpec((HID, 1), lambda j: (0, 0)),
            pl.BlockSpec((HID, 1), lambda j: (0, 0)),
        ],
        out_specs=pl.BlockSpec((1, BE), lambda j: (0, j)),
        out_shape=jax.ShapeDtypeStruct((1, E), _F32),
    )(eaT, r2r, w0r, w0e, b0c, w1c)


def kernel(x, h, v, edge_attr, edge_index,
           W_phi0, b_phi0, W_phi1, W_vel0, b_vel0, W_vel1, b_vel1):
    N = x.shape[0]
    E, D_EDGE = edge_attr.shape
    NL = W_phi0.shape[0]
    NP = ((N + 1 + _NS * 128 - 1) // (_NS * 128)) * (_NS * 128)
    EP = ((E + _NW * 128 - 1) // (_NW * 128)) * (_NW * 128)
    epad = jnp.full((EP - E,), N, jnp.int32)
    row = jnp.concatenate([edge_index[0], epad])
    col = jnp.concatenate([edge_index[1], epad])
    rc = jnp.bitwise_or(row, jnp.left_shift(col, 14))

    xp = jnp.zeros((3, NP), _F32).at[:, :N].set(x.T)
    vp = jnp.zeros((3, NP), _F32).at[:, :N].set(v.T)
    eaT = jnp.zeros((D_EDGE, EP), _F32).at[:, :E].set(edge_attr.T)

    w0r = W_phi0[:, 0:1, :].transpose(0, 2, 1)
    w0e = W_phi0[:, 1:, :].transpose(0, 2, 1)
    b0c = b_phi0[:, :, None]
    w1c = W_phi1
    wv0t = W_vel0.transpose(0, 2, 1)
    bv0c = b_vel0[:, :, None]
    bv1c = b_vel1[:, :, None]

    vs_all = _tc_vscale(vp, wv0t, bv0c, W_vel1, bv1c)

    gather_k = _make_sc_gather(NP, EP)
    count_k = _make_sc_count(NP, EP)
    scatupd_k = _make_sc_scatter_update(NP, EP)

    cnt_inv = count_k(row)
    for l in range(NL):
        r2 = gather_k(rc, xp[0], xp[1], xp[2])
        t = _tc_edge_mlp(eaT, r2.reshape(1, EP),
                         w0r[l], w0e[l], b0c[l], w1c[l])
        xp = scatupd_k(rc, t.reshape(EP), cnt_inv,
                       xp[0], xp[1], xp[2], vp, vs_all[l])

    xout = xp[:, :N].T
    return xout, h


# speedup vs baseline: 15.0336x; 1.2570x over previous
import functools

import jax
import jax.numpy as jnp
from jax import lax
from jax.experimental import pallas as pl
from jax.experimental.pallas import tpu as pltpu
from jax.experimental.pallas import tpu_sc as plsc

_F32 = jnp.float32
_NC = 2
_NS = 16
_NW = _NC * _NS
_L = 16
_GR = 8


def _mesh():
    return plsc.VectorSubcoreMesh(
        core_axis_name="c", subcore_axis_name="s",
        num_cores=_NC, num_subcores=_NS)


_SC_PARAMS = pltpu.CompilerParams(needs_layout_passes=False)


@functools.lru_cache(maxsize=None)
def _make_sc_gather(NP, E):
    EC = E // _NW

    @functools.partial(
        pl.kernel,
        mesh=_mesh(),
        compiler_params=_SC_PARAMS,
        out_type=jax.ShapeDtypeStruct((E,), _F32),
        scratch_types=(
            [pltpu.VMEM((NP,), _F32)] * 3
            + [pltpu.VMEM((EC,), jnp.int32)]
            + [pltpu.VMEM((EC,), _F32)]
            + [pltpu.SemaphoreType.DMA]
        ),
    )
    def k(rc, x0, x1, x2, r2,
          x0v, x1v, x2v, rcv, r2v, semi):
        wid = lax.axis_index("c") * _NS + lax.axis_index("s")
        base = wid * EC
        cps = [
            pltpu.async_copy(rc.at[pl.ds(base, EC)], rcv, semi),
            pltpu.async_copy(x0, x0v, semi),
            pltpu.async_copy(x1, x1v, semi),
            pltpu.async_copy(x2, x2v, semi),
        ]
        for cp in cps:
            cp.wait()

        @plsc.parallel_loop(0, EC, step=_L, unroll=8)
        def body(i):
            s = pl.ds(i, _L)
            rcw = rcv[s]
            rv = rcw & 0x3FFF
            cv = rcw >> 14
            ax = plsc.load_gather(x0v, [rv]) - plsc.load_gather(x0v, [cv])
            ay = plsc.load_gather(x1v, [rv]) - plsc.load_gather(x1v, [cv])
            az = plsc.load_gather(x2v, [rv]) - plsc.load_gather(x2v, [cv])
            r2v[s] = ax * ax + ay * ay + az * az

        pltpu.sync_copy(r2v, r2.at[pl.ds(base, EC)])

    return k


@functools.lru_cache(maxsize=None)
def _make_sc_count(NP, EP):
    EC = EP // _NS
    SL = NP // _NS

    @functools.partial(
        pl.kernel,
        mesh=_mesh(),
        compiler_params=_SC_PARAMS,
        out_type=jax.ShapeDtypeStruct((NP,), _F32),
        scratch_types=(
            [pltpu.VMEM((EC,), jnp.int32)]
            + [pltpu.VMEM((NP,), _F32)]
            + [pltpu.VMEM((SL,), _F32)] * 2
            + [pltpu.VMEM_SHARED((_GR, 1, NP), _F32)]
        ),
    )
    def k(row, out, rowv, acc, tmps, sums, stage):
        cid = lax.axis_index("c")
        sid = lax.axis_index("s")
        base = sid * EC
        pltpu.sync_copy(row.at[pl.ds(base, EC)], rowv)

        @plsc.parallel_loop(0, NP, step=_L, unroll=4)
        def zacc(i):
            acc[pl.ds(i, _L)] = jnp.zeros((_L,), _F32)

        onev = jnp.full((_L,), 1.0, _F32)

        def body(i, carry):
            plsc.addupdate_scatter(acc, [rowv[pl.ds(i * _L, _L)]], onev)
            return carry

        lax.fori_loop(0, EC // _L, body, 0, unroll=8)

        nbase = sid * SL

        @plsc.parallel_loop(0, SL, step=_L, unroll=4)
        def zsum(i):
            sums[pl.ds(i, _L)] = jnp.zeros((_L,), _F32)

        for r in range(_NS // _GR):
            @pl.when((sid >= r * _GR) & (sid < (r + 1) * _GR))
            def _stage():
                pltpu.sync_copy(acc, stage.at[sid - r * _GR, 0])

            plsc.subcore_barrier()
            for g in range(_GR):
                pltpu.sync_copy(stage.at[g, 0, pl.ds(nbase, SL)], tmps)

                def add_body(i2, c2):
                    s2 = pl.ds(i2 * _L, _L)
                    sums[s2] = sums[s2] + tmps[s2]
                    return c2

                lax.fori_loop(0, SL // _L, add_body, 0, unroll=4)
            plsc.subcore_barrier()

        @plsc.parallel_loop(0, SL, step=_L, unroll=4)
        def inv(i):
            s = pl.ds(i, _L)
            sums[s] = 1.0 / jnp.maximum(sums[s], 1.0)

        @pl.when(cid == 0)
        def _w():
            pltpu.sync_copy(sums, out.at[pl.ds(nbase, SL)])

    return k


@functools.lru_cache(maxsize=None)
def _make_sc_scatter_update(NP, EP):
    EC = EP // _NS
    EH = EC // 2
    SL = NP // _NS
    GR = 4

    @functools.partial(
        pl.kernel,
        mesh=_mesh(),
        compiler_params=_SC_PARAMS,
        out_type=jax.ShapeDtypeStruct((3, NP), _F32),
        scratch_types=(
            [pltpu.VMEM((EC,), jnp.int32)]
            + [pltpu.VMEM((EC,), _F32)]
            + [pltpu.VMEM((NP,), _F32)] * 6
            + [pltpu.VMEM((4, SL), _F32)] * 2
            + [pltpu.VMEM((3, SL), _F32)] * 2
            + [pltpu.VMEM((SL,), _F32)] * 2
            + [pltpu.SemaphoreType.DMA] * 2
            + [pltpu.VMEM_SHARED((GR, 4, NP), _F32)]
        ),
    )
    def k(rc, t, cinv, x0, x1, x2, vp, vsl, out,
          rcv, tv, x0v, x1v, x2v, ax, ay, az, tmpv, sumv, vv, ov, vsv, civ,
          semi, sems, stage):
        cid = lax.axis_index("c")
        sid = lax.axis_index("s")
        base = sid * EC
        nbase = sid * SL
        cpi = [
            pltpu.async_copy(rc.at[pl.ds(base, EH)], rcv.at[pl.ds(0, EH)], semi),
            pltpu.async_copy(rc.at[pl.ds(base + EH, EH)], rcv.at[pl.ds(EH, EH)], semi),
            pltpu.async_copy(t.at[pl.ds(base, EH)], tv.at[pl.ds(0, EH)], semi),
            pltpu.async_copy(t.at[pl.ds(base + EH, EH)], tv.at[pl.ds(EH, EH)], semi),
            pltpu.async_copy(x0, x0v, semi),
            pltpu.async_copy(x1, x1v, semi),
            pltpu.async_copy(x2, x2v, semi),
        ]
        cps = [
            pltpu.async_copy(vp.at[:, pl.ds(nbase, SL)], vv, sems),
            pltpu.async_copy(vsl.at[pl.ds(nbase, SL)], vsv, sems),
            pltpu.async_copy(cinv.at[pl.ds(nbase, SL)], civ, sems),
        ]

        @plsc.parallel_loop(0, NP, step=_L, unroll=4)
        def zacc(i):
            s = pl.ds(i, _L)
            zv = jnp.zeros((_L,), _F32)
            ax[s] = zv
            ay[s] = zv
            az[s] = zv

        for cp in cpi:
            cp.wait()

        def body(i, carry):
            s = pl.ds(i * _L, _L)
            rcw = rcv[s]
            rv = rcw & 0x3FFF
            cv = rcw >> 14
            tw = tv[s]
            mxw = (plsc.load_gather(x0v, [rv]) - plsc.load_gather(x0v, [cv])) * tw
            myw = (plsc.load_gather(x1v, [rv]) - plsc.load_gather(x1v, [cv])) * tw
            mzw = (plsc.load_gather(x2v, [rv]) - plsc.load_gather(x2v, [cv])) * tw
            plsc.addupdate_scatter(ax, [rv], mxw)
            plsc.addupdate_scatter(ay, [rv], myw)
            plsc.addupdate_scatter(az, [rv], mzw)
            return carry

        lax.fori_loop(0, EC // _L, body, 0, unroll=8)

        @plsc.parallel_loop(0, SL, step=_L, unroll=4)
        def zsum(i):
            s = pl.ds(i, _L)
            zv = jnp.zeros((_L,), _F32)
            for c in range(3):
                sumv[c, s] = zv

        accs = (ax, ay, az)
        for r in range(_NS // GR):
            @pl.when((sid >= r * GR) & (sid < (r + 1) * GR))
            def _stage():
                slot = sid - r * GR
                for c in range(3):
                    pltpu.sync_copy(accs[c], stage.at[slot, c])

            plsc.subcore_barrier()
            for g in range(GR):
                pltpu.sync_copy(stage.at[g, :, pl.ds(nbase, SL)], tmpv)

                def add_body(i2, c2):
                    s2 = pl.ds(i2 * _L, _L)
                    for c in range(3):
                        sumv[c, s2] = sumv[c, s2] + tmpv[c, s2]
                    return c2

                lax.fori_loop(0, SL // _L, add_body, 0, unroll=4)
            plsc.subcore_barrier()

        for cp in cps:
            cp.wait()

        xslices = (x0v, x1v, x2v)

        @plsc.parallel_loop(0, SL, step=_L, unroll=4)
        def upd(i):
            s = pl.ds(i, _L)
            sx = pl.ds(nbase + i, _L)
            iv = civ[s]
            vs = vsv[s]
            for c in range(3):
                ov[c, s] = xslices[c][sx] + sumv[c, s] * iv + vv[c, s] * vs

        @pl.when(cid == 0)
        def _w():
            pltpu.sync_copy(ov, out.at[:, pl.ds(nbase, SL)])

    return k


def _tc_vscale(vp, wv0t, bv0c, wv1, bv1c):
    NL, HID = wv0t.shape[0], wv0t.shape[1]
    NP = vp.shape[1]

    def body(vp_ref, w0_ref, b0_ref, w1_ref, b1_ref, o_ref):
        v0 = vp_ref[0:1, :]
        v1 = vp_ref[1:2, :]
        v2 = vp_ref[2:3, :]
        vn = jnp.sqrt(v0 * v0 + v1 * v1 + v2 * v2)
        hid = w0_ref[0] * vn + b0_ref[0]
        hid = hid * jax.nn.sigmoid(hid)
        vs = jnp.sum(hid * w1_ref[0], axis=0, keepdims=True) + b1_ref[0]
        o_ref[0] = vs

    return pl.pallas_call(
        body,
        grid=(NL,),
        in_specs=[
            pl.BlockSpec((3, NP), lambda l: (0, 0)),
            pl.BlockSpec((1, HID, 1), lambda l: (l, 0, 0)),
            pl.BlockSpec((1, HID, 1), lambda l: (l, 0, 0)),
            pl.BlockSpec((1, HID, 1), lambda l: (l, 0, 0)),
            pl.BlockSpec((1, 1, 1), lambda l: (l, 0, 0)),
        ],
        out_specs=pl.BlockSpec((1, 1, NP), lambda l: (l, 0, 0)),
        out_shape=jax.ShapeDtypeStruct((NL, 1, NP), _F32),
    )(vp, wv0t, bv0c, wv1, bv1c).reshape(NL, NP)


def _tc_edge_mlp(eaT, r2r, w0r, w0e, b0c, w1c):
    D_EDGE, E = eaT.shape
    HID = w0e.shape[0]
    BE = 8192
    G = E // BE

    def body(ea_ref, r2_ref, w0r_ref, w0e_ref, b0_ref, w1_ref, t_ref):
        rad = jnp.sqrt(r2_ref[...]).astype(jnp.bfloat16)
        hid = lax.dot_general(
            w0e_ref[...], ea_ref[...],
            (((1,), (0,)), ((), ())),
            preferred_element_type=_F32).astype(jnp.bfloat16)
        hid = hid + w0r_ref[...] * rad + b0_ref[...]
        hid = hid * jax.nn.sigmoid(hid)
        tp = lax.dot_general(
            w1_ref[...], hid,
            (((0,), (0,)), ((), ())),
            preferred_element_type=_F32)
        t_ref[...] = jnp.tanh(tp)

    return pl.pallas_call(
        body,
        grid=(G,),
        in_specs=[
            pl.BlockSpec((D_EDGE, BE), lambda j: (0, j)),
            pl.BlockSpec((1, BE), lambda j: (0, j)),
            pl.BlockSpec((HID, 1), lambda j: (0, 0)),
            pl.BlockSpec((HID, D_EDGE), lambda j: (0, 0)),
            pl.BlockSpec((HID, 1), lambda j: (0, 0)),
            pl.BlockSpec((HID, 1), lambda j: (0, 0)),
        ],
        out_specs=pl.BlockSpec((1, BE), lambda j: (0, j)),
        out_shape=jax.ShapeDtypeStruct((1, E), _F32),
    )(eaT, r2r, w0r.astype(jnp.bfloat16), w0e.astype(jnp.bfloat16),
      b0c.astype(jnp.bfloat16), w1c.astype(jnp.bfloat16))


def kernel(x, h, v, edge_attr, edge_index,
           W_phi0, b_phi0, W_phi1, W_vel0, b_vel0, W_vel1, b_vel1):
    N = x.shape[0]
    E, D_EDGE = edge_attr.shape
    NL = W_phi0.shape[0]
    NP = ((N + 1 + _NS * 128 - 1) // (_NS * 128)) * (_NS * 128)
    EP = ((E + _NW * 256 - 1) // (_NW * 256)) * (_NW * 256)
    epad = jnp.full((EP - E,), N, jnp.int32)
    row = jnp.concatenate([edge_index[0], epad])
    col = jnp.concatenate([edge_index[1], epad])
    rc = jnp.bitwise_or(row, jnp.left_shift(col, 14))

    xp = jnp.zeros((3, NP), _F32).at[:, :N].set(x.T)
    vp = jnp.zeros((3, NP), _F32).at[:, :N].set(v.T)
    eaT = jnp.zeros((D_EDGE, EP), jnp.bfloat16).at[:, :E].set(
        edge_attr.T.astype(jnp.bfloat16))

    w0r = W_phi0[:, 0:1, :].transpose(0, 2, 1)
    w0e = W_phi0[:, 1:, :].transpose(0, 2, 1)
    b0c = b_phi0[:, :, None]
    w1c = W_phi1
    wv0t = W_vel0.transpose(0, 2, 1)
    bv0c = b_vel0[:, :, None]
    bv1c = b_vel1[:, :, None]

    vs_all = _tc_vscale(vp, wv0t, bv0c, W_vel1, bv1c)

    gather_k = _make_sc_gather(NP, EP)
    count_k = _make_sc_count(NP, EP)
    scatupd_k = _make_sc_scatter_update(NP, EP)

    cnt_inv = count_k(row)
    for l in range(NL):
        r2 = gather_k(rc, xp[0], xp[1], xp[2])
        t = _tc_edge_mlp(eaT, r2.reshape(1, EP),
                         w0r[l], w0e[l], b0c[l], w1c[l])
        xp = scatupd_k(rc, t.reshape(EP), cnt_inv,
                       xp[0], xp[1], xp[2], vp, vs_all[l])

    xout = xp[:, :N].T
    return xout, h


# speedup vs baseline: 16.1196x; 1.0722x over previous
import functools

import jax
import jax.numpy as jnp
from jax import lax
from jax.experimental import pallas as pl
from jax.experimental.pallas import tpu as pltpu
from jax.experimental.pallas import tpu_sc as plsc

_F32 = jnp.float32
_NC = 2
_NS = 16
_NW = _NC * _NS
_L = 16
_GR = 8


def _mesh():
    return plsc.VectorSubcoreMesh(
        core_axis_name="c", subcore_axis_name="s",
        num_cores=_NC, num_subcores=_NS)


_SC_PARAMS = pltpu.CompilerParams(needs_layout_passes=False)


@functools.lru_cache(maxsize=None)
def _make_sc_gather(NP, E):
    EC = E // _NW

    @functools.partial(
        pl.kernel,
        mesh=_mesh(),
        compiler_params=_SC_PARAMS,
        out_type=jax.ShapeDtypeStruct((E,), _F32),
        scratch_types=(
            [pltpu.VMEM((NP,), _F32)] * 3
            + [pltpu.VMEM((EC,), jnp.int32)]
            + [pltpu.VMEM((EC,), _F32)]
            + [pltpu.SemaphoreType.DMA]
        ),
    )
    def k(rc, x0, x1, x2, r2,
          x0v, x1v, x2v, rcv, r2v, semi):
        wid = lax.axis_index("c") * _NS + lax.axis_index("s")
        base = wid * EC
        cps = [
            pltpu.async_copy(rc.at[pl.ds(base, EC)], rcv, semi),
            pltpu.async_copy(x0, x0v, semi),
            pltpu.async_copy(x1, x1v, semi),
            pltpu.async_copy(x2, x2v, semi),
        ]
        for cp in cps:
            cp.wait()

        @plsc.parallel_loop(0, EC, step=_L, unroll=8)
        def body(i):
            s = pl.ds(i, _L)
            rcw = rcv[s]
            rv = rcw & 0x3FFF
            cv = rcw >> 14
            ax = plsc.load_gather(x0v, [rv]) - plsc.load_gather(x0v, [cv])
            ay = plsc.load_gather(x1v, [rv]) - plsc.load_gather(x1v, [cv])
            az = plsc.load_gather(x2v, [rv]) - plsc.load_gather(x2v, [cv])
            r2v[s] = ax * ax + ay * ay + az * az

        pltpu.sync_copy(r2v, r2.at[pl.ds(base, EC)])

    return k


@functools.lru_cache(maxsize=None)
def _make_sc_count(NP, EP):
    EC = EP // _NS
    SL = NP // _NS

    @functools.partial(
        pl.kernel,
        mesh=_mesh(),
        compiler_params=_SC_PARAMS,
        out_type=jax.ShapeDtypeStruct((NP,), _F32),
        scratch_types=(
            [pltpu.VMEM((EC,), jnp.int32)]
            + [pltpu.VMEM((NP,), _F32)]
            + [pltpu.VMEM((SL,), _F32)] * 2
            + [pltpu.VMEM_SHARED((_GR, 1, NP), _F32)]
        ),
    )
    def k(row, out, rowv, acc, tmps, sums, stage):
        cid = lax.axis_index("c")
        sid = lax.axis_index("s")
        base = sid * EC
        pltpu.sync_copy(row.at[pl.ds(base, EC)], rowv)

        @plsc.parallel_loop(0, NP, step=_L, unroll=4)
        def zacc(i):
            acc[pl.ds(i, _L)] = jnp.zeros((_L,), _F32)

        onev = jnp.full((_L,), 1.0, _F32)

        def body(i, carry):
            plsc.addupdate_scatter(acc, [rowv[pl.ds(i * _L, _L)]], onev)
            return carry

        lax.fori_loop(0, EC // _L, body, 0, unroll=8)

        nbase = sid * SL

        @plsc.parallel_loop(0, SL, step=_L, unroll=4)
        def zsum(i):
            sums[pl.ds(i, _L)] = jnp.zeros((_L,), _F32)

        for r in range(_NS // _GR):
            @pl.when((sid >= r * _GR) & (sid < (r + 1) * _GR))
            def _stage():
                pltpu.sync_copy(acc, stage.at[sid - r * _GR, 0])

            plsc.subcore_barrier()
            for g in range(_GR):
                pltpu.sync_copy(stage.at[g, 0, pl.ds(nbase, SL)], tmps)

                def add_body(i2, c2):
                    s2 = pl.ds(i2 * _L, _L)
                    sums[s2] = sums[s2] + tmps[s2]
                    return c2

                lax.fori_loop(0, SL // _L, add_body, 0, unroll=4)
            plsc.subcore_barrier()

        @plsc.parallel_loop(0, SL, step=_L, unroll=4)
        def inv(i):
            s = pl.ds(i, _L)
            sums[s] = 1.0 / jnp.maximum(sums[s], 1.0)

        @pl.when(cid == 0)
        def _w():
            pltpu.sync_copy(sums, out.at[pl.ds(nbase, SL)])

    return k


@functools.lru_cache(maxsize=None)
def _make_sc_scatter_update(NP, EP):
    EC = EP // _NS
    EH = EC // 2
    SL = NP // _NS
    GR = 4

    @functools.partial(
        pl.kernel,
        mesh=_mesh(),
        compiler_params=_SC_PARAMS,
        out_type=jax.ShapeDtypeStruct((3, NP), _F32),
        scratch_types=(
            [pltpu.VMEM((EC,), jnp.int32)]
            + [pltpu.VMEM((EC,), _F32)]
            + [pltpu.VMEM((NP,), _F32)] * 6
            + [pltpu.VMEM((4, SL), _F32)] * 2
            + [pltpu.VMEM((3, SL), _F32)] * 2
            + [pltpu.VMEM((SL,), _F32)] * 2
            + [pltpu.SemaphoreType.DMA] * 2
            + [pltpu.VMEM_SHARED((GR, 4, NP), _F32)]
        ),
    )
    def k(rc, t, cinv, x0, x1, x2, vp, vsl, out,
          rcv, tv, x0v, x1v, x2v, ax, ay, az, tmpv, sumv, vv, ov, vsv, civ,
          semi, sems, stage):
        cid = lax.axis_index("c")
        sid = lax.axis_index("s")
        base = sid * EC
        nbase = sid * SL
        cpi = [
            pltpu.async_copy(rc.at[pl.ds(base, EH)], rcv.at[pl.ds(0, EH)], semi),
            pltpu.async_copy(rc.at[pl.ds(base + EH, EH)], rcv.at[pl.ds(EH, EH)], semi),
            pltpu.async_copy(t.at[pl.ds(base, EH)], tv.at[pl.ds(0, EH)], semi),
            pltpu.async_copy(t.at[pl.ds(base + EH, EH)], tv.at[pl.ds(EH, EH)], semi),
            pltpu.async_copy(x0, x0v, semi),
            pltpu.async_copy(x1, x1v, semi),
            pltpu.async_copy(x2, x2v, semi),
        ]
        cps = [
            pltpu.async_copy(vp.at[:, pl.ds(nbase, SL)], vv, sems),
            pltpu.async_copy(vsl.at[pl.ds(nbase, SL)], vsv, sems),
            pltpu.async_copy(cinv.at[pl.ds(nbase, SL)], civ, sems),
        ]

        @plsc.parallel_loop(0, NP, step=_L, unroll=4)
        def zacc(i):
            s = pl.ds(i, _L)
            zv = jnp.zeros((_L,), _F32)
            ax[s] = zv
            ay[s] = zv
            az[s] = zv

        for cp in cpi:
            cp.wait()

        @plsc.parallel_loop(0, EC, step=_L, unroll=8)
        def body(i):
            s = pl.ds(i, _L)
            rcw = rcv[s]
            rv = rcw & 0x3FFF
            cv = rcw >> 14
            tw = tv[s]
            mxw = (plsc.load_gather(x0v, [rv]) - plsc.load_gather(x0v, [cv])) * tw
            myw = (plsc.load_gather(x1v, [rv]) - plsc.load_gather(x1v, [cv])) * tw
            mzw = (plsc.load_gather(x2v, [rv]) - plsc.load_gather(x2v, [cv])) * tw
            plsc.addupdate_scatter(ax, [rv], mxw)
            plsc.addupdate_scatter(ay, [rv], myw)
            plsc.addupdate_scatter(az, [rv], mzw)

        @plsc.parallel_loop(0, SL, step=_L, unroll=4)
        def zsum(i):
            s = pl.ds(i, _L)
            zv = jnp.zeros((_L,), _F32)
            for c in range(3):
                sumv[c, s] = zv

        accs = (ax, ay, az)
        for r in range(_NS // GR):
            @pl.when((sid >= r * GR) & (sid < (r + 1) * GR))
            def _stage():
                slot = sid - r * GR
                for c in range(3):
                    pltpu.sync_copy(accs[c], stage.at[slot, c])

            plsc.subcore_barrier()
            for g in range(GR):
                pltpu.sync_copy(stage.at[g, :, pl.ds(nbase, SL)], tmpv)

                def add_body(i2, c2):
                    s2 = pl.ds(i2 * _L, _L)
                    for c in range(3):
                        sumv[c, s2] = sumv[c, s2] + tmpv[c, s2]
                    return c2

                lax.fori_loop(0, SL // _L, add_body, 0, unroll=4)
            plsc.subcore_barrier()

        for cp in cps:
            cp.wait()

        xslices = (x0v, x1v, x2v)

        @plsc.parallel_loop(0, SL, step=_L, unroll=4)
        def upd(i):
            s = pl.ds(i, _L)
            sx = pl.ds(nbase + i, _L)
            iv = civ[s]
            vs = vsv[s]
            for c in range(3):
                ov[c, s] = xslices[c][sx] + sumv[c, s] * iv + vv[c, s] * vs

        @pl.when(cid == 0)
        def _w():
            pltpu.sync_copy(ov, out.at[:, pl.ds(nbase, SL)])

    return k


def _tc_vscale(vp, wv0t, bv0c, wv1, bv1c):
    NL, HID = wv0t.shape[0], wv0t.shape[1]
    NP = vp.shape[1]

    def body(vp_ref, w0_ref, b0_ref, w1_ref, b1_ref, o_ref):
        v0 = vp_ref[0:1, :]
        v1 = vp_ref[1:2, :]
        v2 = vp_ref[2:3, :]
        vn = jnp.sqrt(v0 * v0 + v1 * v1 + v2 * v2)
        hid = w0_ref[0] * vn + b0_ref[0]
        hid = hid * jax.nn.sigmoid(hid)
        vs = jnp.sum(hid * w1_ref[0], axis=0, keepdims=True) + b1_ref[0]
        o_ref[0] = vs

    return pl.pallas_call(
        body,
        grid=(NL,),
        in_specs=[
            pl.BlockSpec((3, NP), lambda l: (0, 0)),
            pl.BlockSpec((1, HID, 1), lambda l: (l, 0, 0)),
            pl.BlockSpec((1, HID, 1), lambda l: (l, 0, 0)),
            pl.BlockSpec((1, HID, 1), lambda l: (l, 0, 0)),
            pl.BlockSpec((1, 1, 1), lambda l: (l, 0, 0)),
        ],
        out_specs=pl.BlockSpec((1, 1, NP), lambda l: (l, 0, 0)),
        out_shape=jax.ShapeDtypeStruct((NL, 1, NP), _F32),
    )(vp, wv0t, bv0c, wv1, bv1c).reshape(NL, NP)


def _tc_edge_mlp(eaT, r2r, w0r, w0e, b0c, w1c):
    D_EDGE, E = eaT.shape
    HID = w0e.shape[0]
    BE = 8192
    G = E // BE

    def body(ea_ref, r2_ref, w0r_ref, w0e_ref, b0_ref, w1_ref, t_ref):
        rad = jnp.sqrt(r2_ref[...]).astype(jnp.bfloat16)
        hid = lax.dot_general(
            w0e_ref[...], ea_ref[...],
            (((1,), (0,)), ((), ())),
            preferred_element_type=_F32).astype(jnp.bfloat16)
        hid = hid + w0r_ref[...] * rad + b0_ref[...]
        hid = hid * jax.nn.sigmoid(hid)
        tp = lax.dot_general(
            w1_ref[...], hid,
            (((0,), (0,)), ((), ())),
            preferred_element_type=_F32)
        t_ref[...] = jnp.tanh(tp)

    return pl.pallas_call(
        body,
        grid=(G,),
        in_specs=[
            pl.BlockSpec((D_EDGE, BE), lambda j: (0, j)),
            pl.BlockSpec((1, BE), lambda j: (0, j)),
            pl.BlockSpec((HID, 1), lambda j: (0, 0)),
            pl.BlockSpec((HID, D_EDGE), lambda j: (0, 0)),
            pl.BlockSpec((HID, 1), lambda j: (0, 0)),
            pl.BlockSpec((HID, 1), lambda j: (0, 0)),
        ],
        out_specs=pl.BlockSpec((1, BE), lambda j: (0, j)),
        out_shape=jax.ShapeDtypeStruct((1, E), _F32),
    )(eaT, r2r, w0r.astype(jnp.bfloat16), w0e.astype(jnp.bfloat16),
      b0c.astype(jnp.bfloat16), w1c.astype(jnp.bfloat16))


def kernel(x, h, v, edge_attr, edge_index,
           W_phi0, b_phi0, W_phi1, W_vel0, b_vel0, W_vel1, b_vel1):
    N = x.shape[0]
    E, D_EDGE = edge_attr.shape
    NL = W_phi0.shape[0]
    NP = ((N + 1 + _NS * 128 - 1) // (_NS * 128)) * (_NS * 128)
    EP = ((E + _NW * 256 - 1) // (_NW * 256)) * (_NW * 256)
    epad = jnp.full((EP - E,), N, jnp.int32)
    row = jnp.concatenate([edge_index[0], epad])
    col = jnp.concatenate([edge_index[1], epad])
    rc = jnp.bitwise_or(row, jnp.left_shift(col, 14))

    xp = jnp.zeros((3, NP), _F32).at[:, :N].set(x.T)
    vp = jnp.zeros((3, NP), _F32).at[:, :N].set(v.T)
    eaT = jnp.zeros((D_EDGE, EP), jnp.bfloat16).at[:, :E].set(
        edge_attr.T.astype(jnp.bfloat16))

    w0r = W_phi0[:, 0:1, :].transpose(0, 2, 1)
    w0e = W_phi0[:, 1:, :].transpose(0, 2, 1)
    b0c = b_phi0[:, :, None]
    w1c = W_phi1
    wv0t = W_vel0.transpose(0, 2, 1)
    bv0c = b_vel0[:, :, None]
    bv1c = b_vel1[:, :, None]

    vs_all = _tc_vscale(vp, wv0t, bv0c, W_vel1, bv1c)

    gather_k = _make_sc_gather(NP, EP)
    count_k = _make_sc_count(NP, EP)
    scatupd_k = _make_sc_scatter_update(NP, EP)

    cnt_inv = count_k(row)
    for l in range(NL):
        r2 = gather_k(rc, xp[0], xp[1], xp[2])
        t = _tc_edge_mlp(eaT, r2.reshape(1, EP),
                         w0r[l], w0e[l], b0c[l], w1c[l])
        xp = scatupd_k(rc, t.reshape(EP), cnt_inv,
                       xp[0], xp[1], xp[2], vp, vs_all[l])

    xout = xp[:, :N].T
    return xout, h
